# Initial kernel scaffold; baseline (speedup 1.0000x reference)
import jax
import jax.numpy as jnp
from jax.experimental import pallas as pl


def kernel(cart, neighlist, shifts, center_factor, neigh_factor, species, params):
    raise NotImplementedError("write your pallas kernel here")


# speedup vs baseline: 1.0035x; 1.0035x over previous
import functools

import jax
import jax.numpy as jnp
import numpy as np
from jax.experimental import pallas as pl

NWAVE = 8
NANG = 9
NCON = 64
CUTOFF = 4.0
ITER_LOOP = 3
INDEX_L = np.array([0, 1, 1, 1, 2, 2, 2, 2, 2])


def _silu(x):
    return x * jax.nn.sigmoid(x)


def _ln(x):
    m = jnp.mean(x, axis=-1, keepdims=True)
    v = jnp.var(x, axis=-1, keepdims=True)
    return (x - m) / jnp.sqrt(v + 1e-5)


def _nnmod(x, p):
    h = _silu(jnp.dot(x, p['W1']) + p['b1'])
    h = _ln(h)
    h = _silu(jnp.dot(h, p['W2']) + p['b2'])
    h = _ln(h)
    return jnp.dot(h, p['Wo']) + p['bo']


def _orbital_kernel(dv_ref, nc_ref, nf_ref, orb_ref):
    dv = dv_ref[...]
    nc = nc_ref[...]
    nf = nf_ref[...]
    x = dv[:, 0:1]
    y = dv[:, 1:2]
    z = dv[:, 2:3]
    r2 = x * x + y * y + z * z
    dist = jnp.sqrt(r2)
    c1 = 0.4886025119029199
    c2 = 1.0925484305920792
    sph = jnp.concatenate([
        jnp.full_like(x, 0.28209479177387814),
        c1 * y, c1 * z, c1 * x,
        c2 * x * y, c2 * y * z,
        0.31539156525252005 * (3.0 * z * z - r2),
        c2 * x * z,
        0.5462742152960396 * (x * x - y * y)], axis=1)
    t = 0.5 * jnp.cos(dist * (np.pi / CUTOFF)) + 0.5
    cut = nf * t * t
    w1 = nc[:, 0:NWAVE]
    w2 = nc[:, NWAVE:2 * NWAVE]
    w3 = nc[:, 2 * NWAVE:3 * NWAVE]
    radial = jnp.exp(-jnp.square(w2 * (dist - w3)))
    a = cut * radial * w1
    orb = sph[:, :, None] * a[:, None, :]
    orb_ref[...] = orb.reshape(orb.shape[0], NANG * NWAVE)


def _orbital_pallas(dv, nc, nf):
    E = dv.shape[0]
    B = 2048
    grid = (E // B,)
    return pl.pallas_call(
        _orbital_kernel,
        grid=grid,
        in_specs=[
            pl.BlockSpec((B, 3), lambda i: (i, 0)),
            pl.BlockSpec((B, 24), lambda i: (i, 0)),
            pl.BlockSpec((B, 1), lambda i: (i, 0)),
        ],
        out_specs=pl.BlockSpec((B, NANG * NWAVE), lambda i: (i, 0)),
        out_shape=jax.ShapeDtypeStruct((E, NANG * NWAVE), jnp.float32),
    )(dv, nc, nf)


def kernel(cart, neighlist, shifts, center_factor, neigh_factor, species, params):
    idx_c = neighlist[0]
    idx_n = neighlist[1]
    distvec = cart[idx_n] - cart[idx_c] + shifts
    center_coeff = _nnmod(species, params['center'])
    neigh_coeff = _nnmod(species, params['neigh'])
    nc24 = neigh_coeff[idx_n]
    dist = jnp.linalg.norm(distvec, axis=1)
    t = 0.5 * jnp.cos(dist * (np.pi / CUTOFF)) + 0.5
    cut_distances = neigh_factor * t * t
    orbital = _orbital_pallas(distvec, nc24, neigh_factor[:, None])
    orbital = orbital.reshape(-1, NANG, NWAVE)
    cc = params['contracted_coeff'][:, INDEX_L]
    center_orbital = jnp.zeros((cart.shape[0], NANG, NWAVE), cart.dtype).at[idx_c].add(orbital)
    contracted = jnp.einsum('ikj,kjm->ikm', center_orbital, cc[0])
    density = jnp.einsum('ikm,ikm,im->im', contracted, contracted, center_coeff)
    for tt in range(ITER_LOOP):
        iter_coeff = _nnmod(density, params['iter'][tt])
        weight_orbital = jnp.einsum('ij,ikj->ikj', iter_coeff[idx_n], orbital) \
            + jnp.einsum('ikj,i->ikj', center_orbital[idx_n], cut_distances)
        center_orbital = center_orbital.at[idx_c].add(weight_orbital)
        contracted = jnp.einsum('ikj,kjm->ikm', center_orbital, cc[tt + 1])
        density = density + jnp.einsum('ikm,ikm,im->im', contracted, contracted, center_coeff)
    output = _nnmod(density, params['out'])
    return jnp.einsum('ij,i->', output, center_factor)


# speedup vs baseline: 33.1937x; 33.0767x over previous
import functools

import jax
import jax.numpy as jnp
import numpy as np
from jax import lax
from jax.experimental import pallas as pl
from jax.experimental.pallas import tpu as pltpu
from jax.experimental.pallas import tpu_sc as plsc

N = 10000
E = 320000
NWAVE = 8
NANG = 9
NCON = 64
CUTOFF = 4.0
ITER_LOOP = 3
INDEX_L = np.array([0, 1, 1, 1, 2, 2, 2, 2, 2])

ROW = 80
ERB = 32
KCH = ROW // 16

NC_ = 2
NS_ = 16
NWK = NC_ * NS_
PER_W = E // NWK
B = 80
NBATCH = PER_W // B
RPT = N // NS_

_mesh = plsc.VectorSubcoreMesh(core_axis_name="c", subcore_axis_name="s")


def _silu(x):
    return x * jax.nn.sigmoid(x)


def _ln(x):
    m = jnp.mean(x, axis=-1, keepdims=True)
    v = jnp.var(x, axis=-1, keepdims=True)
    return (x - m) / jnp.sqrt(v + 1e-5)


_SC_PARAMS = pltpu.CompilerParams(use_tc_tiling_on_sc=False,
                                  needs_layout_passes=False)


@functools.partial(
    pl.kernel,
    compiler_params=_SC_PARAMS,
    out_type=[
        jax.ShapeDtypeStruct((E, ERB), jnp.float32),
        jax.ShapeDtypeStruct((E, 16), jnp.float32),
    ],
    mesh=_mesh,
    scratch_types=[
        pltpu.VMEM((B,), jnp.int32),
        pltpu.VMEM((B,), jnp.int32),
        pltpu.VMEM((B, ERB), jnp.float32),
        pltpu.VMEM((B, 16), jnp.float32),
        pltpu.SemaphoreType.DMA,
        pltpu.SemaphoreType.DMA,
    ],
)
def _sc_gather(ntab, cart16, idxn3, idxc3, gtn_out, gtc_out,
               idxn_v, idxc_v, gtn_v, gtc_v, s1, s2):
    c = lax.axis_index("c")
    s = lax.axis_index("s")
    wid = s * NC_ + c

    def batch(b, carry):
        base = wid * PER_W + b * B
        pltpu.sync_copy(idxn3.at[wid, b], idxn_v)
        pltpu.sync_copy(idxc3.at[wid, b], idxc_v)
        d1 = pltpu.async_copy(ntab.at[idxn_v], gtn_v, s1)
        d2 = pltpu.async_copy(cart16.at[idxc_v], gtc_v, s2)
        d1.wait()
        d2.wait()
        pltpu.sync_copy(gtn_v, gtn_out.at[pl.ds(base, B)])
        pltpu.sync_copy(gtc_v, gtc_out.at[pl.ds(base, B)])
        return carry

    lax.fori_loop(0, NBATCH, batch, 0)


def _sc_pass_body(first, er_hbm, idxn3, idxc3, co_hbm, ic_hbm, out_hbm,
                  idxn_v, idxc_v, er_v, co_v, ic_v, w_v, z_v, accum,
                  sg, si):
    c = lax.axis_index("c")
    s = lax.axis_index("s")
    wid = s * NC_ + c
    lane = lax.iota(jnp.int32, 16)
    j8 = lane & 7
    aa_idx = j8 + 16
    sph_idx = [(ci * 16 + lane) >> 3 for ci in range(KCH)]
    cut_idx = jnp.full((16,), 24, jnp.int32)
    zeros16 = jnp.zeros((16,), jnp.float32)

    def zrow(i, carry):
        for ci in range(KCH):
            z_v[i, pl.ds(ci * 16, 16)] = zeros16
        return carry

    lax.fori_loop(0, RPT // 5, zrow, 0)
    for k in range(5):
        pltpu.sync_copy(z_v, accum.at[pl.ds(s * RPT + k * (RPT // 5), RPT // 5)])
    plsc.subcore_barrier()

    def batch(b, carry):
        base = wid * PER_W + b * B
        pltpu.sync_copy(idxc3.at[wid, b], idxc_v)
        pltpu.sync_copy(er_hbm.at[pl.ds(base, B)], er_v)
        if not first:
            pltpu.sync_copy(idxn3.at[wid, b], idxn_v)
            dg = pltpu.async_copy(co_hbm.at[idxn_v], co_v, sg)
            di = pltpu.async_copy(ic_hbm.at[idxn_v], ic_v, si)
            dg.wait()
            di.wait()

        def edge(e, carry2):
            efull = jnp.full((16,), e, jnp.int32)
            aab = plsc.load_gather(er_v, [efull, aa_idx])
            if first:
                for ci in range(KCH):
                    sphb = plsc.load_gather(er_v, [efull, sph_idx[ci]])
                    w_v[e, pl.ds(ci * 16, 16)] = sphb * aab
            else:
                icb = plsc.load_gather(ic_v, [efull, j8])
                bb = icb * aab
                cut = plsc.load_gather(er_v, [efull, cut_idx])
                for ci in range(KCH):
                    sphb = plsc.load_gather(er_v, [efull, sph_idx[ci]])
                    cov = co_v[e, pl.ds(ci * 16, 16)]
                    w_v[e, pl.ds(ci * 16, 16)] = bb * sphb + cut * cov
            return carry2

        lax.fori_loop(0, B, edge, 0)
        pltpu.sync_copy(w_v, accum.at[idxc_v], add=True)
        return carry

    lax.fori_loop(0, NBATCH, batch, 0)
    plsc.subcore_barrier()
    for k in range(5):
        lo = s * RPT + k * (RPT // 5)
        pltpu.sync_copy(accum.at[pl.ds(lo, RPT // 5)],
                        out_hbm.at[c, pl.ds(lo, RPT // 5)])


_PASS_SCRATCH = [
    pltpu.VMEM((B,), jnp.int32),
    pltpu.VMEM((B,), jnp.int32),
    pltpu.VMEM((B, ERB), jnp.float32),
    pltpu.VMEM((B, ROW), jnp.float32),
    pltpu.VMEM((B, 16), jnp.float32),
    pltpu.VMEM((B, ROW), jnp.float32),
    pltpu.VMEM((RPT // 5, ROW), jnp.float32),
    pltpu.VMEM_SHARED((N, ROW), jnp.float32),
    pltpu.SemaphoreType.DMA,
    pltpu.SemaphoreType.DMA,
]

_OUT_PARTIAL = jax.ShapeDtypeStruct((NC_, N, ROW), jnp.float32)


@functools.partial(pl.kernel, out_type=_OUT_PARTIAL, mesh=_mesh,
                   compiler_params=_SC_PARAMS, scratch_types=_PASS_SCRATCH)
def _sc_pass0(er_hbm, idxc3, out_hbm,
              idxn_v, idxc_v, er_v, co_v, ic_v, w_v, z_v, accum, sg, si):
    _sc_pass_body(True, er_hbm, None, idxc3, None, None, out_hbm,
                  idxn_v, idxc_v, er_v, co_v, ic_v, w_v, z_v, accum, sg, si)


@functools.partial(pl.kernel, out_type=_OUT_PARTIAL, mesh=_mesh,
                   compiler_params=_SC_PARAMS, scratch_types=_PASS_SCRATCH)
def _sc_pass_iter(er_hbm, idxn3, idxc3, co_hbm, ic_hbm, out_hbm,
                  idxn_v, idxc_v, er_v, co_v, ic_v, w_v, z_v, accum, sg, si):
    _sc_pass_body(False, er_hbm, idxn3, idxc3, co_hbm, ic_hbm, out_hbm,
                  idxn_v, idxc_v, er_v, co_v, ic_v, w_v, z_v, accum, sg, si)


def _full_spec(shape):
    return pl.BlockSpec(shape, lambda i: tuple(0 for _ in shape))


def _mlp(x, w1, b1, w2, b2, wo, bo):
    h = _silu(jnp.dot(x, w1, preferred_element_type=jnp.float32) + b1)
    h = _ln(h)
    h = _silu(jnp.dot(h, w2, preferred_element_type=jnp.float32) + b2)
    h = _ln(h)
    return jnp.dot(h, wo, preferred_element_type=jnp.float32) + bo


def _tc_species_kernel(sp, cb4, cw1, cb1, cw2, cb2, cwo, cbo,
                       nw1, nb1, nw2, nb2, nwo, nbo, ccf_ref, ntab_ref):
    sp_ = sp[...]
    h = _silu(sp_ * cw1[...] + cb1[...])
    h = _ln(h)
    h = _silu(jnp.dot(h, cw2[...], preferred_element_type=jnp.float32) + cb2[...])
    h = _ln(h)
    ccf_ref[...] = jnp.dot(h, cwo[...], preferred_element_type=jnp.float32) + cbo[...]
    g = _silu(sp_ * nw1[...] + nb1[...])
    g = _ln(g)
    g = _silu(jnp.dot(g, nw2[...], preferred_element_type=jnp.float32) + nb2[...])
    g = _ln(g)
    nc = jnp.dot(g, nwo[...], preferred_element_type=jnp.float32) + nbo[...]
    ntab_ref[...] = jnp.concatenate(
        [nc, cb4[...][:, :3], jnp.zeros_like(nc[:, :5])], axis=1)


def _tc_edges_kernel(gtn, gtc, sh, nf, er_ref):
    ncg_ = gtn[...]
    dv = ncg_[:, 24:27] - gtc[...][:, :3] + sh[...][:, :3]
    x = dv[:, 0:1]
    y = dv[:, 1:2]
    z = dv[:, 2:3]
    r2 = x * x + y * y + z * z
    dist = jnp.sqrt(r2)
    c1 = 0.4886025119029199
    c2 = 1.0925484305920792
    sph = jnp.concatenate([
        jnp.full_like(x, 0.28209479177387814),
        c1 * y, c1 * z, c1 * x,
        c2 * x * y, c2 * y * z,
        0.31539156525252005 * (3.0 * z * z - r2),
        c2 * x * z,
        0.5462742152960396 * (x * x - y * y)], axis=1)
    t = 0.5 * jnp.cos(dist * (np.pi / CUTOFF)) + 0.5
    cut = nf[...] * t * t
    w1 = ncg_[:, 0:NWAVE]
    w2 = ncg_[:, NWAVE:2 * NWAVE]
    w3 = ncg_[:, 2 * NWAVE:3 * NWAVE]
    radial = jnp.exp(-jnp.square(w2 * (dist - w3)))
    aa = cut * radial * w1
    z7 = jnp.zeros_like(ncg_[:, :7])
    er_ref[...] = jnp.concatenate([sph, z7, aa, cut, z7], axis=1)


def _tc_iter_a_kernel(p0, p1, co_prev, w, ccf, den_prev, co_ref, den_ref):
    co = co_prev[...] + p0[0] + p1[0]
    co_ref[...] = co
    cf = jnp.dot(co, w[...], preferred_element_type=jnp.float32)
    acc = None
    for k in range(NANG):
        blk = cf[:, k * NCON:(k + 1) * NCON]
        sq = blk * blk
        acc = sq if acc is None else acc + sq
    den_ref[...] = den_prev[...] + acc * ccf[...]


def _tc_iter_b_kernel(den, w1, b1, w2, b2, wo, bo, it_ref):
    o = _mlp(den[...], w1[...], b1[...], w2[...], b2[...], wo[...], bo[...])
    it_ref[...] = jnp.concatenate([o, jnp.zeros_like(o)], axis=1)


def _tc_final_kernel(den, w1, b1, w2, b2, wo, bo, cf2, out_ref):
    i = pl.program_id(0)
    o = _mlp(den[...], w1[...], b1[...], w2[...], b2[...], wo[...], bo[...])
    part = jnp.sum(o * cf2[...])

    @pl.when(i == 0)
    def _():
        out_ref[...] = jnp.zeros_like(out_ref)

    out_ref[...] += jnp.full((1, 1), 1.0, jnp.float32) * part


BN_N = 2000
BN_E = 4000


def _species_call(sp, cart4, pc, pn):
    specs = [pl.BlockSpec((BN_N, 1), lambda i: (i, 0)),
             pl.BlockSpec((BN_N, 4), lambda i: (i, 0))]
    args = [sp, cart4]
    for p in (pc, pn):
        for nm, bshape in (('W1', None), ('b1', None), ('W2', None),
                           ('b2', None), ('Wo', None), ('bo', None)):
            a = p[nm]
            if a.ndim == 1:
                a = a[None, :]
            specs.append(_full_spec(a.shape))
            args.append(a)
    return pl.pallas_call(
        _tc_species_kernel,
        grid=(N // BN_N,),
        in_specs=specs,
        out_specs=[pl.BlockSpec((BN_N, NCON), lambda i: (i, 0)),
                   pl.BlockSpec((BN_N, ERB), lambda i: (i, 0))],
        out_shape=[jax.ShapeDtypeStruct((N, NCON), jnp.float32),
                   jax.ShapeDtypeStruct((N, ERB), jnp.float32)],
    )(*args)


def _edges_call(gtn, gtc, sh4, nf2):
    return pl.pallas_call(
        _tc_edges_kernel,
        grid=(E // BN_E,),
        in_specs=[pl.BlockSpec((BN_E, ERB), lambda i: (i, 0)),
                  pl.BlockSpec((BN_E, 16), lambda i: (i, 0)),
                  pl.BlockSpec((BN_E, 4), lambda i: (i, 0)),
                  pl.BlockSpec((BN_E, 1), lambda i: (i, 0))],
        out_specs=pl.BlockSpec((BN_E, ERB), lambda i: (i, 0)),
        out_shape=jax.ShapeDtypeStruct((E, ERB), jnp.float32),
    )(gtn, gtc, sh4, nf2)


def _iter_a_call(p, co_prev, w, ccf, den_prev):
    return pl.pallas_call(
        _tc_iter_a_kernel,
        grid=(N // BN_N,),
        in_specs=[pl.BlockSpec((1, BN_N, ROW), lambda i: (0, i, 0)),
                  pl.BlockSpec((1, BN_N, ROW), lambda i: (1, i, 0)),
                  pl.BlockSpec((BN_N, ROW), lambda i: (i, 0)),
                  _full_spec(w.shape),
                  pl.BlockSpec((BN_N, NCON), lambda i: (i, 0)),
                  pl.BlockSpec((BN_N, NCON), lambda i: (i, 0))],
        out_specs=[pl.BlockSpec((BN_N, ROW), lambda i: (i, 0)),
                   pl.BlockSpec((BN_N, NCON), lambda i: (i, 0))],
        out_shape=[jax.ShapeDtypeStruct((N, ROW), jnp.float32),
                   jax.ShapeDtypeStruct((N, NCON), jnp.float32)],
    )(p, p, co_prev, w, ccf, den_prev)


def _iter_b_call(den, pit):
    args = [den]
    specs = [pl.BlockSpec((BN_N, NCON), lambda i: (i, 0))]
    for nm in ('W1', 'b1', 'W2', 'b2', 'Wo', 'bo'):
        a = pit[nm]
        if a.ndim == 1:
            a = a[None, :]
        specs.append(_full_spec(a.shape))
        args.append(a)
    return pl.pallas_call(
        _tc_iter_b_kernel,
        grid=(N // BN_N,),
        in_specs=specs,
        out_specs=pl.BlockSpec((BN_N, 16), lambda i: (i, 0)),
        out_shape=jax.ShapeDtypeStruct((N, 16), jnp.float32),
    )(*args)


def _final_call(den, pout, cf2):
    args = [den]
    specs = [pl.BlockSpec((BN_N, NCON), lambda i: (i, 0))]
    for nm in ('W1', 'b1', 'W2', 'b2', 'Wo', 'bo'):
        a = pout[nm]
        if a.ndim == 1:
            a = a[None, :]
        specs.append(_full_spec(a.shape))
        args.append(a)
    args.append(cf2)
    specs.append(pl.BlockSpec((BN_N, 1), lambda i: (i, 0)))
    return pl.pallas_call(
        _tc_final_kernel,
        grid=(N // BN_N,),
        in_specs=specs,
        out_specs=pl.BlockSpec((1, 1), lambda i: (0, 0)),
        out_shape=jax.ShapeDtypeStruct((1, 1), jnp.float32),
    )(*args)


def kernel(cart, neighlist, shifts, center_factor, neigh_factor, species, params):
    f32 = jnp.float32
    idx_c = neighlist[0].astype(jnp.int32)
    idx_n = neighlist[1].astype(jnp.int32)
    idxn3 = idx_n.reshape(NWK, NBATCH, B)
    idxc3 = idx_c.reshape(NWK, NBATCH, B)
    cart4 = jnp.pad(cart.astype(f32), ((0, 0), (0, 1)))
    cart16 = jnp.pad(cart.astype(f32), ((0, 0), (0, 13)))
    sh4 = jnp.pad(shifts.astype(f32), ((0, 0), (0, 1)))
    nf2 = neigh_factor.astype(f32)[:, None]
    cf2 = center_factor.astype(f32)[:, None]

    ccf, ntab = _species_call(species.astype(f32), cart4,
                              params['center'], params['neigh'])
    gtn, gtc = _sc_gather(ntab, cart16, idxn3, idxc3)
    er = _edges_call(gtn, gtc, sh4, nf2)

    cc_full = params['contracted_coeff'][:, INDEX_L]
    ws = []
    for t in range(ITER_LOOP + 1):
        w = jnp.zeros((ROW, NANG * NCON), f32)
        for k in range(NANG):
            w = w.at[k * NWAVE:(k + 1) * NWAVE, k * NCON:(k + 1) * NCON].set(cc_full[t, k])
        ws.append(w)

    p = _sc_pass0(er, idxc3)
    zeros_row = jnp.zeros((N, ROW), f32)
    zeros_den = jnp.zeros((N, NCON), f32)
    co, den = _iter_a_call(p, zeros_row, ws[0], ccf, zeros_den)
    for t in range(ITER_LOOP):
        it16 = _iter_b_call(den, params['iter'][t])
        p = _sc_pass_iter(er, idxn3, idxc3, co, it16)
        co, den = _iter_a_call(p, co, ws[t + 1], ccf, den)
    res = _final_call(den, params['out'], cf2)
    return res[0, 0]


# speedup vs baseline: 41.2519x; 1.2428x over previous
import functools

import jax
import jax.numpy as jnp
import numpy as np
from jax import lax
from jax.experimental import pallas as pl
from jax.experimental.pallas import tpu as pltpu
from jax.experimental.pallas import tpu_sc as plsc

N = 10000
E = 320000
NWAVE = 8
NANG = 9
NCON = 64
CUTOFF = 4.0
ITER_LOOP = 3
INDEX_L = np.array([0, 1, 1, 1, 2, 2, 2, 2, 2])

ROW = 80
ERB = 32
KCH = ROW // 16

NC_ = 2
NS_ = 16
NWK = NC_ * NS_
PER_W = E // NWK
B = 80
NBATCH = PER_W // B
RPT = N // NS_

_mesh = plsc.VectorSubcoreMesh(core_axis_name="c", subcore_axis_name="s")


def _silu(x):
    return x * jax.nn.sigmoid(x)


def _ln(x):
    m = jnp.mean(x, axis=-1, keepdims=True)
    v = jnp.var(x, axis=-1, keepdims=True)
    return (x - m) / jnp.sqrt(v + 1e-5)


_SC_PARAMS = pltpu.CompilerParams(use_tc_tiling_on_sc=False,
                                  needs_layout_passes=False)


@functools.partial(
    pl.kernel,
    compiler_params=_SC_PARAMS,
    out_type=[
        jax.ShapeDtypeStruct((E, ERB), jnp.float32),
        jax.ShapeDtypeStruct((E, 16), jnp.float32),
    ],
    mesh=_mesh,
    scratch_types=[
        pltpu.VMEM((B,), jnp.int32),
        pltpu.VMEM((B,), jnp.int32),
        pltpu.VMEM((B, ERB), jnp.float32),
        pltpu.VMEM((B, 16), jnp.float32),
        pltpu.SemaphoreType.DMA,
        pltpu.SemaphoreType.DMA,
    ],
)
def _sc_gather(ntab, cart16, idxn1, idxc1, gtn_out, gtc_out,
               idxn_v, idxc_v, gtn_v, gtc_v, s1, s2):
    c = lax.axis_index("c")
    s = lax.axis_index("s")
    wid = s * NC_ + c

    def batch(b, carry):
        base = wid * PER_W + b * B
        pltpu.sync_copy(idxn1.at[pl.ds(base, B)], idxn_v)
        pltpu.sync_copy(idxc1.at[pl.ds(base, B)], idxc_v)
        d1 = pltpu.async_copy(ntab.at[idxn_v], gtn_v, s1)
        d2 = pltpu.async_copy(cart16.at[idxc_v], gtc_v, s2)
        d1.wait()
        d2.wait()
        pltpu.sync_copy(gtn_v, gtn_out.at[pl.ds(base, B)])
        pltpu.sync_copy(gtc_v, gtc_out.at[pl.ds(base, B)])
        return carry

    lax.fori_loop(0, NBATCH, batch, 0)


def _sc_pass_body(first, er_hbm, idxn1, idxc1, co_hbm, ic_hbm, out_hbm,
                  idxn_v, idxc_v, er_v, co_v, ic_v, w_v, z_v, accum,
                  sg, si):
    c = lax.axis_index("c")
    s = lax.axis_index("s")
    wid = s * NC_ + c
    lane = lax.iota(jnp.int32, 16)
    j8 = lane & 7
    aa_idx = j8 + 16
    sph_idx = [(ci * 16 + lane) >> 3 for ci in range(KCH)]
    cut_idx = jnp.full((16,), 24, jnp.int32)
    zeros16 = jnp.zeros((16,), jnp.float32)

    def zrow(i, carry):
        for ci in range(KCH):
            z_v[i, pl.ds(ci * 16, 16)] = zeros16
        return carry

    lax.fori_loop(0, RPT // 5, zrow, 0)
    for k in range(5):
        pltpu.sync_copy(z_v, accum.at[pl.ds(s * RPT + k * (RPT // 5), RPT // 5)])
    plsc.subcore_barrier()

    def batch(b, carry):
        base = wid * PER_W + b * B
        pltpu.sync_copy(idxc1.at[pl.ds(base, B)], idxc_v)
        pltpu.sync_copy(er_hbm.at[pl.ds(base, B)], er_v)
        if not first:
            pltpu.sync_copy(idxn1.at[pl.ds(base, B)], idxn_v)
            dg = pltpu.async_copy(co_hbm.at[idxn_v], co_v, sg)
            di = pltpu.async_copy(ic_hbm.at[idxn_v], ic_v, si)
            dg.wait()
            di.wait()

        def edge(e, carry2):
            efull = jnp.full((16,), e, jnp.int32)
            aab = plsc.load_gather(er_v, [efull, aa_idx])
            if first:
                for ci in range(KCH):
                    sphb = plsc.load_gather(er_v, [efull, sph_idx[ci]])
                    w_v[e, pl.ds(ci * 16, 16)] = sphb * aab
            else:
                icb = plsc.load_gather(ic_v, [efull, j8])
                bb = icb * aab
                cut = plsc.load_gather(er_v, [efull, cut_idx])
                for ci in range(KCH):
                    sphb = plsc.load_gather(er_v, [efull, sph_idx[ci]])
                    cov = co_v[e, pl.ds(ci * 16, 16)]
                    w_v[e, pl.ds(ci * 16, 16)] = bb * sphb + cut * cov
            return carry2

        lax.fori_loop(0, B, edge, 0)
        pltpu.sync_copy(w_v, accum.at[idxc_v], add=True)
        return carry

    lax.fori_loop(0, NBATCH, batch, 0)
    plsc.subcore_barrier()
    for k in range(5):
        lo = s * RPT + k * (RPT // 5)
        pltpu.sync_copy(accum.at[pl.ds(lo, RPT // 5)],
                        out_hbm.at[c, pl.ds(lo, RPT // 5)])


_PASS_SCRATCH = [
    pltpu.VMEM((B,), jnp.int32),
    pltpu.VMEM((B,), jnp.int32),
    pltpu.VMEM((B, ERB), jnp.float32),
    pltpu.VMEM((B, ROW), jnp.float32),
    pltpu.VMEM((B, 16), jnp.float32),
    pltpu.VMEM((B, ROW), jnp.float32),
    pltpu.VMEM((RPT // 5, ROW), jnp.float32),
    pltpu.VMEM_SHARED((N, ROW), jnp.float32),
    pltpu.SemaphoreType.DMA,
    pltpu.SemaphoreType.DMA,
]

_OUT_PARTIAL = jax.ShapeDtypeStruct((NC_, N, ROW), jnp.float32)


@functools.partial(pl.kernel, out_type=_OUT_PARTIAL, mesh=_mesh,
                   compiler_params=_SC_PARAMS, scratch_types=_PASS_SCRATCH)
def _sc_pass0(er_hbm, idxc1, out_hbm,
              idxn_v, idxc_v, er_v, co_v, ic_v, w_v, z_v, accum, sg, si):
    _sc_pass_body(True, er_hbm, None, idxc1, None, None, out_hbm,
                  idxn_v, idxc_v, er_v, co_v, ic_v, w_v, z_v, accum, sg, si)


@functools.partial(pl.kernel, out_type=_OUT_PARTIAL, mesh=_mesh,
                   compiler_params=_SC_PARAMS, scratch_types=_PASS_SCRATCH)
def _sc_pass_iter(er_hbm, idxn1, idxc1, co_hbm, ic_hbm, out_hbm,
                  idxn_v, idxc_v, er_v, co_v, ic_v, w_v, z_v, accum, sg, si):
    _sc_pass_body(False, er_hbm, idxn1, idxc1, co_hbm, ic_hbm, out_hbm,
                  idxn_v, idxc_v, er_v, co_v, ic_v, w_v, z_v, accum, sg, si)


def _full_spec(shape):
    return pl.BlockSpec(shape, lambda i: tuple(0 for _ in shape))


def _mlp(x, w1, b1, w2, b2, wo, bo):
    h = _silu(jnp.dot(x, w1, preferred_element_type=jnp.float32) + b1)
    h = _ln(h)
    h = _silu(jnp.dot(h, w2, preferred_element_type=jnp.float32) + b2)
    h = _ln(h)
    return jnp.dot(h, wo, preferred_element_type=jnp.float32) + bo


def _tc_species_kernel(sp, cb4, cw1, cb1, cw2, cb2, cwo, cbo,
                       nw1, nb1, nw2, nb2, nwo, nbo, ccf_ref, ntab_ref):
    sp_ = sp[...]
    h = _silu(sp_ * cw1[...] + cb1[...])
    h = _ln(h)
    h = _silu(jnp.dot(h, cw2[...], preferred_element_type=jnp.float32) + cb2[...])
    h = _ln(h)
    ccf_ref[...] = jnp.dot(h, cwo[...], preferred_element_type=jnp.float32) + cbo[...]
    g = _silu(sp_ * nw1[...] + nb1[...])
    g = _ln(g)
    g = _silu(jnp.dot(g, nw2[...], preferred_element_type=jnp.float32) + nb2[...])
    g = _ln(g)
    nc = jnp.dot(g, nwo[...], preferred_element_type=jnp.float32) + nbo[...]
    ntab_ref[...] = jnp.concatenate(
        [nc, cb4[...][:, :3], jnp.zeros_like(nc[:, :5])], axis=1)


def _tc_edges_kernel(gtn, gtc, sh, nf, er_ref):
    gt = gtn[...].T
    gc = gtc[...].T
    sht = sh[...].T
    nft = nf[...].T
    dv = gt[24:27, :] - gc[0:3, :] + sht
    x = dv[0:1, :]
    y = dv[1:2, :]
    z = dv[2:3, :]
    r2 = x * x + y * y + z * z
    dist = jnp.sqrt(r2)
    c1 = 0.4886025119029199
    c2 = 1.0925484305920792
    sph = jnp.concatenate([
        jnp.full_like(x, 0.28209479177387814),
        c1 * y, c1 * z, c1 * x,
        c2 * x * y, c2 * y * z,
        0.31539156525252005 * (3.0 * z * z - r2),
        c2 * x * z,
        0.5462742152960396 * (x * x - y * y)], axis=0)
    t = 0.5 * jnp.cos(dist * (np.pi / CUTOFF)) + 0.5
    cut = nft * t * t
    w1 = gt[0:NWAVE, :]
    w2 = gt[NWAVE:2 * NWAVE, :]
    w3 = gt[2 * NWAVE:3 * NWAVE, :]
    radial = jnp.exp(-jnp.square(w2 * (dist - w3)))
    aa = cut * radial * w1
    z7 = jnp.zeros_like(gt[0:7, :])
    er_ref[...] = jnp.concatenate([sph, z7, aa, cut, z7], axis=0).T


def _tc_iter_a0_kernel(p0, p1, w, ccf, co_ref, den_ref):
    co = p0[0] + p1[0]
    co_ref[...] = co
    cf = jnp.dot(co, w[...], preferred_element_type=jnp.float32)
    acc = None
    for k in range(NANG):
        blk = cf[:, k * NCON:(k + 1) * NCON]
        sq = blk * blk
        acc = sq if acc is None else acc + sq
    den_ref[...] = acc * ccf[...]


def _tc_iter_a_kernel(p0, p1, co_prev, w, ccf, den_prev, co_ref, den_ref):
    co = co_prev[...] + p0[0] + p1[0]
    co_ref[...] = co
    cf = jnp.dot(co, w[...], preferred_element_type=jnp.float32)
    acc = None
    for k in range(NANG):
        blk = cf[:, k * NCON:(k + 1) * NCON]
        sq = blk * blk
        acc = sq if acc is None else acc + sq
    den_ref[...] = den_prev[...] + acc * ccf[...]


def _tc_iter_b_kernel(den, w1, b1, w2, b2, wo, bo, it_ref):
    o = _mlp(den[...], w1[...], b1[...], w2[...], b2[...], wo[...], bo[...])
    it_ref[...] = jnp.concatenate([o, jnp.zeros_like(o)], axis=1)


def _tc_final_kernel(den, w1, b1, w2, b2, wo, bo, cf2, out_ref):
    i = pl.program_id(0)
    o = _mlp(den[...], w1[...], b1[...], w2[...], b2[...], wo[...], bo[...])
    part = jnp.sum(o * cf2[...])

    @pl.when(i == 0)
    def _():
        out_ref[...] = jnp.zeros_like(out_ref)

    out_ref[...] += jnp.full((1, 1), 1.0, jnp.float32) * part


BN_N = 2000
BN_E = 2560


def _species_call(sp, cart4, pc, pn):
    specs = [pl.BlockSpec((BN_N, 1), lambda i: (i, 0)),
             pl.BlockSpec((BN_N, 4), lambda i: (i, 0))]
    args = [sp, cart4]
    for p in (pc, pn):
        for nm, bshape in (('W1', None), ('b1', None), ('W2', None),
                           ('b2', None), ('Wo', None), ('bo', None)):
            a = p[nm]
            if a.ndim == 1:
                a = a[None, :]
            specs.append(_full_spec(a.shape))
            args.append(a)
    return pl.pallas_call(
        _tc_species_kernel,
        grid=(N // BN_N,),
        in_specs=specs,
        out_specs=[pl.BlockSpec((BN_N, NCON), lambda i: (i, 0)),
                   pl.BlockSpec((BN_N, ERB), lambda i: (i, 0))],
        out_shape=[jax.ShapeDtypeStruct((N, NCON), jnp.float32),
                   jax.ShapeDtypeStruct((N, ERB), jnp.float32)],
    )(*args)


def _edges_call(gtn, gtc, sh3, nf2):
    return pl.pallas_call(
        _tc_edges_kernel,
        grid=(E // BN_E,),
        in_specs=[pl.BlockSpec((BN_E, ERB), lambda i: (i, 0)),
                  pl.BlockSpec((BN_E, 16), lambda i: (i, 0)),
                  pl.BlockSpec((BN_E, 3), lambda i: (i, 0)),
                  pl.BlockSpec((BN_E, 1), lambda i: (i, 0))],
        out_specs=pl.BlockSpec((BN_E, ERB), lambda i: (i, 0)),
        out_shape=jax.ShapeDtypeStruct((E, ERB), jnp.float32),
    )(gtn, gtc, sh3, nf2)


def _iter_a0_call(p, w, ccf):
    return pl.pallas_call(
        _tc_iter_a0_kernel,
        grid=(N // BN_N,),
        in_specs=[pl.BlockSpec((1, BN_N, ROW), lambda i: (0, i, 0)),
                  pl.BlockSpec((1, BN_N, ROW), lambda i: (1, i, 0)),
                  _full_spec(w.shape),
                  pl.BlockSpec((BN_N, NCON), lambda i: (i, 0))],
        out_specs=[pl.BlockSpec((BN_N, ROW), lambda i: (i, 0)),
                   pl.BlockSpec((BN_N, NCON), lambda i: (i, 0))],
        out_shape=[jax.ShapeDtypeStruct((N, ROW), jnp.float32),
                   jax.ShapeDtypeStruct((N, NCON), jnp.float32)],
    )(p, p, w, ccf)


def _iter_a_call(p, co_prev, w, ccf, den_prev):
    return pl.pallas_call(
        _tc_iter_a_kernel,
        grid=(N // BN_N,),
        in_specs=[pl.BlockSpec((1, BN_N, ROW), lambda i: (0, i, 0)),
                  pl.BlockSpec((1, BN_N, ROW), lambda i: (1, i, 0)),
                  pl.BlockSpec((BN_N, ROW), lambda i: (i, 0)),
                  _full_spec(w.shape),
                  pl.BlockSpec((BN_N, NCON), lambda i: (i, 0)),
                  pl.BlockSpec((BN_N, NCON), lambda i: (i, 0))],
        out_specs=[pl.BlockSpec((BN_N, ROW), lambda i: (i, 0)),
                   pl.BlockSpec((BN_N, NCON), lambda i: (i, 0))],
        out_shape=[jax.ShapeDtypeStruct((N, ROW), jnp.float32),
                   jax.ShapeDtypeStruct((N, NCON), jnp.float32)],
    )(p, p, co_prev, w, ccf, den_prev)


def _iter_b_call(den, pit):
    args = [den]
    specs = [pl.BlockSpec((BN_N, NCON), lambda i: (i, 0))]
    for nm in ('W1', 'b1', 'W2', 'b2', 'Wo', 'bo'):
        a = pit[nm]
        if a.ndim == 1:
            a = a[None, :]
        specs.append(_full_spec(a.shape))
        args.append(a)
    return pl.pallas_call(
        _tc_iter_b_kernel,
        grid=(N // BN_N,),
        in_specs=specs,
        out_specs=pl.BlockSpec((BN_N, 16), lambda i: (i, 0)),
        out_shape=jax.ShapeDtypeStruct((N, 16), jnp.float32),
    )(*args)


def _final_call(den, pout, cf2):
    args = [den]
    specs = [pl.BlockSpec((BN_N, NCON), lambda i: (i, 0))]
    for nm in ('W1', 'b1', 'W2', 'b2', 'Wo', 'bo'):
        a = pout[nm]
        if a.ndim == 1:
            a = a[None, :]
        specs.append(_full_spec(a.shape))
        args.append(a)
    args.append(cf2)
    specs.append(pl.BlockSpec((BN_N, 1), lambda i: (i, 0)))
    return pl.pallas_call(
        _tc_final_kernel,
        grid=(N // BN_N,),
        in_specs=specs,
        out_specs=pl.BlockSpec((1, 1), lambda i: (0, 0)),
        out_shape=jax.ShapeDtypeStruct((1, 1), jnp.float32),
    )(*args)


def kernel(cart, neighlist, shifts, center_factor, neigh_factor, species, params):
    f32 = jnp.float32
    idx_c = neighlist[0].astype(jnp.int32)
    idx_n = neighlist[1].astype(jnp.int32)
    cart4 = jnp.pad(cart.astype(f32), ((0, 0), (0, 1)))
    cart16 = jnp.pad(cart.astype(f32), ((0, 0), (0, 13)))
    nf2 = neigh_factor.astype(f32)[:, None]
    cf2 = center_factor.astype(f32)[:, None]

    ccf, ntab = _species_call(species.astype(f32), cart4,
                              params['center'], params['neigh'])
    gtn, gtc = _sc_gather(ntab, cart16, idx_n, idx_c)
    er = _edges_call(gtn, gtc, shifts.astype(f32), nf2)

    cc_full = params['contracted_coeff'][:, INDEX_L]
    ws = []
    for t in range(ITER_LOOP + 1):
        w = jnp.zeros((ROW, NANG * NCON), f32)
        for k in range(NANG):
            w = w.at[k * NWAVE:(k + 1) * NWAVE, k * NCON:(k + 1) * NCON].set(cc_full[t, k])
        ws.append(w)

    p = _sc_pass0(er, idx_c)
    co, den = _iter_a0_call(p, ws[0], ccf)
    for t in range(ITER_LOOP):
        it16 = _iter_b_call(den, params['iter'][t])
        p = _sc_pass_iter(er, idx_n, idx_c, co, it16)
        co, den = _iter_a_call(p, co, ws[t + 1], ccf, den)
    res = _final_call(den, params['out'], cf2)
    return res[0, 0]


# speedup vs baseline: 50.4756x; 1.2236x over previous
import functools

import jax
import jax.numpy as jnp
import numpy as np
from jax import lax
from jax.experimental import pallas as pl
from jax.experimental.pallas import tpu as pltpu
from jax.experimental.pallas import tpu_sc as plsc

N = 10000
E = 320000
NWAVE = 8
NANG = 9
NCON = 64
CUTOFF = 4.0
ITER_LOOP = 3
INDEX_L = np.array([0, 1, 1, 1, 2, 2, 2, 2, 2])

ROW = 80
ERB = 32
KCH = ROW // 16

NC_ = 2
NS_ = 16
NWK = NC_ * NS_
PER_W = E // NWK
B = 80
NBATCH = PER_W // B
RPT = N // NS_

_mesh = plsc.VectorSubcoreMesh(core_axis_name="c", subcore_axis_name="s")


def _silu(x):
    return x * jax.nn.sigmoid(x)


def _ln(x):
    m = jnp.mean(x, axis=-1, keepdims=True)
    v = jnp.var(x, axis=-1, keepdims=True)
    return (x - m) / jnp.sqrt(v + 1e-5)


_SC_PARAMS = pltpu.CompilerParams(use_tc_tiling_on_sc=False,
                                  needs_layout_passes=False)


@functools.partial(
    pl.kernel,
    compiler_params=_SC_PARAMS,
    out_type=[
        jax.ShapeDtypeStruct((E, ERB), jnp.float32),
        jax.ShapeDtypeStruct((E, 16), jnp.float32),
    ],
    mesh=_mesh,
    scratch_types=[
        pltpu.VMEM((NBATCH, B), jnp.int32),
        pltpu.VMEM((NBATCH, B), jnp.int32),
        pltpu.VMEM((B, ERB), jnp.float32),
        pltpu.VMEM((B, 16), jnp.float32),
        pltpu.SemaphoreType.DMA,
        pltpu.SemaphoreType.DMA,
    ],
)
def _sc_gather(ntab, cart16, idxn2h, idxc2h, gtn_out, gtc_out,
               idxn_v, idxc_v, gtn_v, gtc_v, s1, s2):
    c = lax.axis_index("c")
    s = lax.axis_index("s")
    wid = s * NC_ + c
    pltpu.sync_copy(idxn2h.at[pl.ds(wid * NBATCH, NBATCH)], idxn_v)
    pltpu.sync_copy(idxc2h.at[pl.ds(wid * NBATCH, NBATCH)], idxc_v)

    def batch(b, carry):
        base = wid * PER_W + b * B
        d1 = pltpu.async_copy(ntab.at[idxn_v.at[b]], gtn_v, s1)
        d2 = pltpu.async_copy(cart16.at[idxc_v.at[b]], gtc_v, s2)
        d1.wait()
        d2.wait()
        pltpu.sync_copy(gtn_v, gtn_out.at[pl.ds(base, B)])
        pltpu.sync_copy(gtc_v, gtc_out.at[pl.ds(base, B)])
        return carry

    lax.fori_loop(0, NBATCH, batch, 0)


def _sc_pass_body(first, er_hbm, idxn2h, idxc2h, co_hbm, ic_hbm, out_hbm,
                  idxn_v, idxc_v, er_v, co_v, ic_v, w_v, z_v, accum,
                  sg, si, se):
    c = lax.axis_index("c")
    s = lax.axis_index("s")
    wid = s * NC_ + c
    pltpu.sync_copy(idxc2h.at[pl.ds(wid * NBATCH, NBATCH)], idxc_v)
    if not first:
        pltpu.sync_copy(idxn2h.at[pl.ds(wid * NBATCH, NBATCH)], idxn_v)
    lane = lax.iota(jnp.int32, 16)
    j8 = lane & 7
    aa_idx = j8 + 16
    sph_idx = [(ci * 16 + lane) >> 3 for ci in range(KCH)]
    cut_idx = jnp.full((16,), 24, jnp.int32)
    zeros16 = jnp.zeros((16,), jnp.float32)

    def zrow(i, carry):
        for ci in range(KCH):
            z_v[i, pl.ds(ci * 16, 16)] = zeros16
        return carry

    lax.fori_loop(0, RPT // 5, zrow, 0)
    for k in range(5):
        pltpu.sync_copy(z_v, accum.at[pl.ds(s * RPT + k * (RPT // 5), RPT // 5)])
    plsc.subcore_barrier()

    def batch(b, carry):
        base = wid * PER_W + b * B
        de = pltpu.async_copy(er_hbm.at[pl.ds(base, B)], er_v, se)
        if not first:
            dg = pltpu.async_copy(co_hbm.at[idxn_v.at[b]], co_v, sg)
            di = pltpu.async_copy(ic_hbm.at[idxn_v.at[b]], ic_v, si)
            dg.wait()
            di.wait()
        de.wait()

        def edge(e, carry2):
            efull = jnp.full((16,), e, jnp.int32)
            aab = plsc.load_gather(er_v, [efull, aa_idx])
            if first:
                for ci in range(KCH):
                    sphb = plsc.load_gather(er_v, [efull, sph_idx[ci]])
                    w_v[e, pl.ds(ci * 16, 16)] = sphb * aab
            else:
                icb = plsc.load_gather(ic_v, [efull, j8])
                bb = icb * aab
                cut = plsc.load_gather(er_v, [efull, cut_idx])
                for ci in range(KCH):
                    sphb = plsc.load_gather(er_v, [efull, sph_idx[ci]])
                    cov = co_v[e, pl.ds(ci * 16, 16)]
                    w_v[e, pl.ds(ci * 16, 16)] = bb * sphb + cut * cov
            return carry2

        lax.fori_loop(0, B, edge, 0)
        pltpu.sync_copy(w_v, accum.at[idxc_v.at[b]], add=True)
        return carry

    lax.fori_loop(0, NBATCH, batch, 0)
    plsc.subcore_barrier()
    for k in range(5):
        lo = s * RPT + k * (RPT // 5)
        pltpu.sync_copy(accum.at[pl.ds(lo, RPT // 5)],
                        out_hbm.at[c, pl.ds(lo, RPT // 5)])


_PASS_SCRATCH = [
    pltpu.VMEM((NBATCH, B), jnp.int32),
    pltpu.VMEM((NBATCH, B), jnp.int32),
    pltpu.VMEM((B, ERB), jnp.float32),
    pltpu.VMEM((B, ROW), jnp.float32),
    pltpu.VMEM((B, 16), jnp.float32),
    pltpu.VMEM((B, ROW), jnp.float32),
    pltpu.VMEM((RPT // 5, ROW), jnp.float32),
    pltpu.VMEM_SHARED((N, ROW), jnp.float32),
    pltpu.SemaphoreType.DMA,
    pltpu.SemaphoreType.DMA,
    pltpu.SemaphoreType.DMA,
]

_OUT_PARTIAL = jax.ShapeDtypeStruct((NC_, N, ROW), jnp.float32)


@functools.partial(pl.kernel, out_type=_OUT_PARTIAL, mesh=_mesh,
                   compiler_params=_SC_PARAMS, scratch_types=_PASS_SCRATCH)
def _sc_pass0(er_hbm, idxc2h, out_hbm,
              idxn_v, idxc_v, er_v, co_v, ic_v, w_v, z_v, accum, sg, si, se):
    _sc_pass_body(True, er_hbm, None, idxc2h, None, None, out_hbm,
                  idxn_v, idxc_v, er_v, co_v, ic_v, w_v, z_v, accum, sg, si, se)


@functools.partial(pl.kernel, out_type=_OUT_PARTIAL, mesh=_mesh,
                   compiler_params=_SC_PARAMS, scratch_types=_PASS_SCRATCH)
def _sc_pass_iter(er_hbm, idxn2h, idxc2h, co_hbm, ic_hbm, out_hbm,
                  idxn_v, idxc_v, er_v, co_v, ic_v, w_v, z_v, accum, sg, si, se):
    _sc_pass_body(False, er_hbm, idxn2h, idxc2h, co_hbm, ic_hbm, out_hbm,
                  idxn_v, idxc_v, er_v, co_v, ic_v, w_v, z_v, accum, sg, si, se)


def _full_spec(shape):
    return pl.BlockSpec(shape, lambda i: tuple(0 for _ in shape))


def _mlp(x, w1, b1, w2, b2, wo, bo):
    h = _silu(jnp.dot(x, w1, preferred_element_type=jnp.float32) + b1)
    h = _ln(h)
    h = _silu(jnp.dot(h, w2, preferred_element_type=jnp.float32) + b2)
    h = _ln(h)
    return jnp.dot(h, wo, preferred_element_type=jnp.float32) + bo


def _tc_species_kernel(sp, cb4, cw1, cb1, cw2, cb2, cwo, cbo,
                       nw1, nb1, nw2, nb2, nwo, nbo, ccf_ref, ntab_ref):
    sp_ = sp[...]
    h = _silu(sp_ * cw1[...] + cb1[...])
    h = _ln(h)
    h = _silu(jnp.dot(h, cw2[...], preferred_element_type=jnp.float32) + cb2[...])
    h = _ln(h)
    ccf_ref[...] = jnp.dot(h, cwo[...], preferred_element_type=jnp.float32) + cbo[...]
    g = _silu(sp_ * nw1[...] + nb1[...])
    g = _ln(g)
    g = _silu(jnp.dot(g, nw2[...], preferred_element_type=jnp.float32) + nb2[...])
    g = _ln(g)
    nc = jnp.dot(g, nwo[...], preferred_element_type=jnp.float32) + nbo[...]
    ntab_ref[...] = jnp.concatenate(
        [nc, cb4[...][:, :3], jnp.zeros_like(nc[:, :5])], axis=1)


def _tc_edges_kernel(gtn, gtc, sh, nf, er_ref):
    gt = gtn[...].T
    gc = gtc[...].T
    sht = sh[...].T
    nft = nf[...].T
    dv = gt[24:27, :] - gc[0:3, :] + sht
    x = dv[0:1, :]
    y = dv[1:2, :]
    z = dv[2:3, :]
    r2 = x * x + y * y + z * z
    dist = jnp.sqrt(r2)
    c1 = 0.4886025119029199
    c2 = 1.0925484305920792
    sph = jnp.concatenate([
        jnp.full_like(x, 0.28209479177387814),
        c1 * y, c1 * z, c1 * x,
        c2 * x * y, c2 * y * z,
        0.31539156525252005 * (3.0 * z * z - r2),
        c2 * x * z,
        0.5462742152960396 * (x * x - y * y)], axis=0)
    t = 0.5 * jnp.cos(dist * (np.pi / CUTOFF)) + 0.5
    cut = nft * t * t
    w1 = gt[0:NWAVE, :]
    w2 = gt[NWAVE:2 * NWAVE, :]
    w3 = gt[2 * NWAVE:3 * NWAVE, :]
    radial = jnp.exp(-jnp.square(w2 * (dist - w3)))
    aa = cut * radial * w1
    z7 = jnp.zeros_like(gt[0:7, :])
    er_ref[...] = jnp.concatenate([sph, z7, aa, cut, z7], axis=0).T


def _tc_iter_a0_kernel(p0, p1, w, ccf, co_ref, den_ref):
    co = p0[0] + p1[0]
    co_ref[...] = co
    cf = jnp.dot(co, w[...], preferred_element_type=jnp.float32)
    acc = None
    for k in range(NANG):
        blk = cf[:, k * NCON:(k + 1) * NCON]
        sq = blk * blk
        acc = sq if acc is None else acc + sq
    den_ref[...] = acc * ccf[...]


def _tc_iter_a_kernel(p0, p1, co_prev, w, ccf, den_prev, co_ref, den_ref):
    co = co_prev[...] + p0[0] + p1[0]
    co_ref[...] = co
    cf = jnp.dot(co, w[...], preferred_element_type=jnp.float32)
    acc = None
    for k in range(NANG):
        blk = cf[:, k * NCON:(k + 1) * NCON]
        sq = blk * blk
        acc = sq if acc is None else acc + sq
    den_ref[...] = den_prev[...] + acc * ccf[...]


def _tc_iter_b_kernel(den, w1, b1, w2, b2, wo, bo, it_ref):
    o = _mlp(den[...], w1[...], b1[...], w2[...], b2[...], wo[...], bo[...])
    it_ref[...] = jnp.concatenate([o, jnp.zeros_like(o)], axis=1)


def _tc_final_kernel(den, w1, b1, w2, b2, wo, bo, cf2, out_ref):
    i = pl.program_id(0)
    o = _mlp(den[...], w1[...], b1[...], w2[...], b2[...], wo[...], bo[...])
    part = jnp.sum(o * cf2[...])

    @pl.when(i == 0)
    def _():
        out_ref[...] = jnp.zeros_like(out_ref)

    out_ref[...] += jnp.full((1, 1), 1.0, jnp.float32) * part


BN_N = 2000
BN_E = 2560


def _species_call(sp, cart4, pc, pn):
    specs = [pl.BlockSpec((BN_N, 1), lambda i: (i, 0)),
             pl.BlockSpec((BN_N, 4), lambda i: (i, 0))]
    args = [sp, cart4]
    for p in (pc, pn):
        for nm, bshape in (('W1', None), ('b1', None), ('W2', None),
                           ('b2', None), ('Wo', None), ('bo', None)):
            a = p[nm]
            if a.ndim == 1:
                a = a[None, :]
            specs.append(_full_spec(a.shape))
            args.append(a)
    return pl.pallas_call(
        _tc_species_kernel,
        grid=(N // BN_N,),
        in_specs=specs,
        out_specs=[pl.BlockSpec((BN_N, NCON), lambda i: (i, 0)),
                   pl.BlockSpec((BN_N, ERB), lambda i: (i, 0))],
        out_shape=[jax.ShapeDtypeStruct((N, NCON), jnp.float32),
                   jax.ShapeDtypeStruct((N, ERB), jnp.float32)],
    )(*args)


def _edges_call(gtn, gtc, sh3, nf2):
    return pl.pallas_call(
        _tc_edges_kernel,
        grid=(E // BN_E,),
        in_specs=[pl.BlockSpec((BN_E, ERB), lambda i: (i, 0)),
                  pl.BlockSpec((BN_E, 16), lambda i: (i, 0)),
                  pl.BlockSpec((BN_E, 3), lambda i: (i, 0)),
                  pl.BlockSpec((BN_E, 1), lambda i: (i, 0))],
        out_specs=pl.BlockSpec((BN_E, ERB), lambda i: (i, 0)),
        out_shape=jax.ShapeDtypeStruct((E, ERB), jnp.float32),
    )(gtn, gtc, sh3, nf2)


def _iter_a0_call(p, w, ccf):
    return pl.pallas_call(
        _tc_iter_a0_kernel,
        grid=(N // BN_N,),
        in_specs=[pl.BlockSpec((1, BN_N, ROW), lambda i: (0, i, 0)),
                  pl.BlockSpec((1, BN_N, ROW), lambda i: (1, i, 0)),
                  _full_spec(w.shape),
                  pl.BlockSpec((BN_N, NCON), lambda i: (i, 0))],
        out_specs=[pl.BlockSpec((BN_N, ROW), lambda i: (i, 0)),
                   pl.BlockSpec((BN_N, NCON), lambda i: (i, 0))],
        out_shape=[jax.ShapeDtypeStruct((N, ROW), jnp.float32),
                   jax.ShapeDtypeStruct((N, NCON), jnp.float32)],
    )(p, p, w, ccf)


def _iter_a_call(p, co_prev, w, ccf, den_prev):
    return pl.pallas_call(
        _tc_iter_a_kernel,
        grid=(N // BN_N,),
        in_specs=[pl.BlockSpec((1, BN_N, ROW), lambda i: (0, i, 0)),
                  pl.BlockSpec((1, BN_N, ROW), lambda i: (1, i, 0)),
                  pl.BlockSpec((BN_N, ROW), lambda i: (i, 0)),
                  _full_spec(w.shape),
                  pl.BlockSpec((BN_N, NCON), lambda i: (i, 0)),
                  pl.BlockSpec((BN_N, NCON), lambda i: (i, 0))],
        out_specs=[pl.BlockSpec((BN_N, ROW), lambda i: (i, 0)),
                   pl.BlockSpec((BN_N, NCON), lambda i: (i, 0))],
        out_shape=[jax.ShapeDtypeStruct((N, ROW), jnp.float32),
                   jax.ShapeDtypeStruct((N, NCON), jnp.float32)],
    )(p, p, co_prev, w, ccf, den_prev)


def _iter_b_call(den, pit):
    args = [den]
    specs = [pl.BlockSpec((BN_N, NCON), lambda i: (i, 0))]
    for nm in ('W1', 'b1', 'W2', 'b2', 'Wo', 'bo'):
        a = pit[nm]
        if a.ndim == 1:
            a = a[None, :]
        specs.append(_full_spec(a.shape))
        args.append(a)
    return pl.pallas_call(
        _tc_iter_b_kernel,
        grid=(N // BN_N,),
        in_specs=specs,
        out_specs=pl.BlockSpec((BN_N, 16), lambda i: (i, 0)),
        out_shape=jax.ShapeDtypeStruct((N, 16), jnp.float32),
    )(*args)


def _final_call(den, pout, cf2):
    args = [den]
    specs = [pl.BlockSpec((BN_N, NCON), lambda i: (i, 0))]
    for nm in ('W1', 'b1', 'W2', 'b2', 'Wo', 'bo'):
        a = pout[nm]
        if a.ndim == 1:
            a = a[None, :]
        specs.append(_full_spec(a.shape))
        args.append(a)
    args.append(cf2)
    specs.append(pl.BlockSpec((BN_N, 1), lambda i: (i, 0)))
    return pl.pallas_call(
        _tc_final_kernel,
        grid=(N // BN_N,),
        in_specs=specs,
        out_specs=pl.BlockSpec((1, 1), lambda i: (0, 0)),
        out_shape=jax.ShapeDtypeStruct((1, 1), jnp.float32),
    )(*args)


def kernel(cart, neighlist, shifts, center_factor, neigh_factor, species, params):
    f32 = jnp.float32
    idx_c = neighlist[0].astype(jnp.int32)
    idx_n = neighlist[1].astype(jnp.int32)
    idxn2h = idx_n.reshape(NWK * NBATCH, B)
    idxc2h = idx_c.reshape(NWK * NBATCH, B)
    cart4 = jnp.pad(cart.astype(f32), ((0, 0), (0, 1)))
    cart16 = jnp.pad(cart.astype(f32), ((0, 0), (0, 13)))
    nf2 = neigh_factor.astype(f32)[:, None]
    cf2 = center_factor.astype(f32)[:, None]

    ccf, ntab = _species_call(species.astype(f32), cart4,
                              params['center'], params['neigh'])
    gtn, gtc = _sc_gather(ntab, cart16, idxn2h, idxc2h)
    er = _edges_call(gtn, gtc, shifts.astype(f32), nf2)

    cc_full = params['contracted_coeff'][:, INDEX_L]
    ws = []
    for t in range(ITER_LOOP + 1):
        w = jnp.zeros((ROW, NANG * NCON), f32)
        for k in range(NANG):
            w = w.at[k * NWAVE:(k + 1) * NWAVE, k * NCON:(k + 1) * NCON].set(cc_full[t, k])
        ws.append(w)

    p = _sc_pass0(er, idxc2h)
    co, den = _iter_a0_call(p, ws[0], ccf)
    for t in range(ITER_LOOP):
        it16 = _iter_b_call(den, params['iter'][t])
        p = _sc_pass_iter(er, idxn2h, idxc2h, co, it16)
        co, den = _iter_a_call(p, co, ws[t + 1], ccf, den)
    res = _final_call(den, params['out'], cf2)
    return res[0, 0]


# speedup vs baseline: 62.2586x; 1.2334x over previous
import functools

import jax
import jax.numpy as jnp
import numpy as np
from jax import lax
from jax.experimental import pallas as pl
from jax.experimental.pallas import tpu as pltpu
from jax.experimental.pallas import tpu_sc as plsc

N = 10000
E = 320000
NWAVE = 8
NANG = 9
NCON = 64
CUTOFF = 4.0
ITER_LOOP = 3
INDEX_L = np.array([0, 1, 1, 1, 2, 2, 2, 2, 2])

ROW = 80
ERB = 32
KCH = ROW // 16

NC_ = 2
NS_ = 16
NWK = NC_ * NS_
PER_W = E // NWK
B = 80
NBATCH = PER_W // B
RPT = N // NS_

_mesh = plsc.VectorSubcoreMesh(core_axis_name="c", subcore_axis_name="s")


def _silu(x):
    return x * jax.nn.sigmoid(x)


def _ln(x):
    m = jnp.mean(x, axis=-1, keepdims=True)
    v = jnp.var(x, axis=-1, keepdims=True)
    return (x - m) / jnp.sqrt(v + 1e-5)


_SC_PARAMS = pltpu.CompilerParams(use_tc_tiling_on_sc=False,
                                  needs_layout_passes=False)


@functools.partial(
    pl.kernel,
    compiler_params=_SC_PARAMS,
    out_type=[
        jax.ShapeDtypeStruct((E, ERB), jnp.float32),
        jax.ShapeDtypeStruct((E, 16), jnp.float32),
    ],
    mesh=_mesh,
    scratch_types=[
        pltpu.VMEM((NBATCH, B), jnp.int32),
        pltpu.VMEM((NBATCH, B), jnp.int32),
        pltpu.VMEM((B, ERB), jnp.float32),
        pltpu.VMEM((B, ERB), jnp.float32),
        pltpu.VMEM((B, 16), jnp.float32),
        pltpu.VMEM((B, 16), jnp.float32),
        pltpu.SemaphoreType.DMA,
        pltpu.SemaphoreType.DMA,
        pltpu.SemaphoreType.DMA,
        pltpu.SemaphoreType.DMA,
        pltpu.SemaphoreType.DMA,
        pltpu.SemaphoreType.DMA,
    ],
)
def _sc_gather(ntab, cart16, idxn2h, idxc2h, gtn_out, gtc_out,
               idxn_v, idxc_v, gtn0, gtn1, gtc0, gtc1,
               sg0, sg1, sc0, sc1, sw0, sw1):
    c = lax.axis_index("c")
    s = lax.axis_index("s")
    wid = s * NC_ + c
    pltpu.sync_copy(idxn2h.at[pl.ds(wid * NBATCH, NBATCH)], idxn_v)
    pltpu.sync_copy(idxc2h.at[pl.ds(wid * NBATCH, NBATCH)], idxc_v)
    gtn = (gtn0, gtn1)
    gtc = (gtc0, gtc1)
    sg = (sg0, sg1)
    sc = (sc0, sc1)
    sw = (sw0, sw1)

    def issue(b, k):
        pltpu.async_copy(ntab.at[idxn_v.at[b]], gtn[k], sg[k])
        pltpu.async_copy(cart16.at[idxc_v.at[b]], gtc[k], sc[k])

    def process(b, k):
        base = wid * PER_W + b * B
        pltpu.make_async_copy(ntab.at[idxn_v.at[b]], gtn[k], sg[k]).wait()
        pltpu.make_async_copy(cart16.at[idxc_v.at[b]], gtc[k], sc[k]).wait()
        pltpu.async_copy(gtn[k], gtn_out.at[pl.ds(base, B)], sw[k])
        pltpu.async_copy(gtc[k], gtc_out.at[pl.ds(base, B)], sw[k])
        pltpu.make_async_copy(gtn[k], gtn_out.at[pl.ds(base, B)], sw[k]).wait()
        pltpu.make_async_copy(gtc[k], gtc_out.at[pl.ds(base, B)], sw[k]).wait()

        @pl.when(b + 2 < NBATCH)
        def _():
            issue(b + 2, k)

    issue(0, 0)
    issue(1, 1)

    def pair(g, carry):
        process(2 * g, 0)
        process(2 * g + 1, 1)
        return carry

    lax.fori_loop(0, NBATCH // 2, pair, 0)
    process(NBATCH - 1, 0)


def _sc_pass_body(first, er_hbm, idxn2h, idxc2h, co_hbm, ic_hbm, out_hbm,
                  idxn_v, idxc_v, er0, er1, co0, co1, ic0, ic1, w0, w1,
                  z_v, accum, se0, se1, sg0, sg1, si0, si1, sx0, sx1):
    c = lax.axis_index("c")
    s = lax.axis_index("s")
    wid = s * NC_ + c
    lane = lax.iota(jnp.int32, 16)
    j8 = lane & 7
    aa_idx = j8 + 16
    sph_idx = [(ci * 16 + lane) >> 3 for ci in range(KCH)]
    cut_idx = jnp.full((16,), 24, jnp.int32)
    zeros16 = jnp.zeros((16,), jnp.float32)
    er = (er0, er1)
    co = (co0, co1)
    ic = (ic0, ic1)
    w = (w0, w1)
    se = (se0, se1)
    sg = (sg0, sg1)
    si = (si0, si1)
    sx = (sx0, sx1)

    pltpu.sync_copy(idxc2h.at[pl.ds(wid * NBATCH, NBATCH)], idxc_v)
    if not first:
        pltpu.sync_copy(idxn2h.at[pl.ds(wid * NBATCH, NBATCH)], idxn_v)

    def zrow(i, carry):
        for ci in range(KCH):
            z_v[i, pl.ds(ci * 16, 16)] = zeros16
        return carry

    lax.fori_loop(0, RPT // 5, zrow, 0)
    for k in range(5):
        pltpu.sync_copy(z_v, accum.at[pl.ds(s * RPT + k * (RPT // 5), RPT // 5)])
    plsc.subcore_barrier()

    def issue(b, k):
        base = wid * PER_W + b * B
        pltpu.async_copy(er_hbm.at[pl.ds(base, B)], er[k], se[k])
        if not first:
            pltpu.async_copy(co_hbm.at[idxn_v.at[b]], co[k], sg[k])
            pltpu.async_copy(ic_hbm.at[idxn_v.at[b]], ic[k], si[k])

    def process(b, k):
        base = wid * PER_W + b * B
        pltpu.make_async_copy(er_hbm.at[pl.ds(base, B)], er[k], se[k]).wait()
        if not first:
            pltpu.make_async_copy(co_hbm.at[idxn_v.at[b]], co[k], sg[k]).wait()
            pltpu.make_async_copy(ic_hbm.at[idxn_v.at[b]], ic[k], si[k]).wait()

        @pl.when(b >= 2)
        def _():
            pltpu.make_async_copy(w[k], accum.at[idxc_v.at[b - 2]], sx[k]).wait()

        er_k, co_k, ic_k, w_k = er[k], co[k], ic[k], w[k]

        def edge(e, carry2):
            efull = jnp.full((16,), e, jnp.int32)
            aab = plsc.load_gather(er_k, [efull, aa_idx])
            if first:
                for ci in range(KCH):
                    sphb = plsc.load_gather(er_k, [efull, sph_idx[ci]])
                    w_k[e, pl.ds(ci * 16, 16)] = sphb * aab
            else:
                icb = plsc.load_gather(ic_k, [efull, j8])
                bb = icb * aab
                cut = plsc.load_gather(er_k, [efull, cut_idx])
                for ci in range(KCH):
                    sphb = plsc.load_gather(er_k, [efull, sph_idx[ci]])
                    cov = co_k[e, pl.ds(ci * 16, 16)]
                    w_k[e, pl.ds(ci * 16, 16)] = bb * sphb + cut * cov
            return carry2

        lax.fori_loop(0, B, edge, 0)
        pltpu.async_copy(w[k], accum.at[idxc_v.at[b]], sx[k], add=True)

        @pl.when(b + 2 < NBATCH)
        def _():
            issue(b + 2, k)

    issue(0, 0)
    issue(1, 1)

    def pair(g, carry):
        process(2 * g, 0)
        process(2 * g + 1, 1)
        return carry

    lax.fori_loop(0, NBATCH // 2, pair, 0)
    process(NBATCH - 1, 0)
    pltpu.make_async_copy(w[0], accum.at[idxc_v.at[NBATCH - 1]], sx[0]).wait()
    pltpu.make_async_copy(w[1], accum.at[idxc_v.at[NBATCH - 2]], sx[1]).wait()
    plsc.subcore_barrier()
    for k in range(5):
        lo = s * RPT + k * (RPT // 5)
        pltpu.sync_copy(accum.at[pl.ds(lo, RPT // 5)],
                        out_hbm.at[c, pl.ds(lo, RPT // 5)])


_PASS_SCRATCH = [
    pltpu.VMEM((NBATCH, B), jnp.int32),
    pltpu.VMEM((NBATCH, B), jnp.int32),
    pltpu.VMEM((B, ERB), jnp.float32),
    pltpu.VMEM((B, ERB), jnp.float32),
    pltpu.VMEM((B, ROW), jnp.float32),
    pltpu.VMEM((B, ROW), jnp.float32),
    pltpu.VMEM((B, 16), jnp.float32),
    pltpu.VMEM((B, 16), jnp.float32),
    pltpu.VMEM((B, ROW), jnp.float32),
    pltpu.VMEM((B, ROW), jnp.float32),
    pltpu.VMEM((RPT // 5, ROW), jnp.float32),
    pltpu.VMEM_SHARED((N, ROW), jnp.float32),
] + [pltpu.SemaphoreType.DMA] * 8

_OUT_PARTIAL = jax.ShapeDtypeStruct((NC_, N, ROW), jnp.float32)


@functools.partial(pl.kernel, out_type=_OUT_PARTIAL, mesh=_mesh,
                   compiler_params=_SC_PARAMS, scratch_types=_PASS_SCRATCH)
def _sc_pass0(er_hbm, idxc2h, out_hbm,
              idxn_v, idxc_v, er0, er1, co0, co1, ic0, ic1, w0, w1,
              z_v, accum, se0, se1, sg0, sg1, si0, si1, sx0, sx1):
    _sc_pass_body(True, er_hbm, None, idxc2h, None, None, out_hbm,
                  idxn_v, idxc_v, er0, er1, co0, co1, ic0, ic1, w0, w1,
                  z_v, accum, se0, se1, sg0, sg1, si0, si1, sx0, sx1)


@functools.partial(pl.kernel, out_type=_OUT_PARTIAL, mesh=_mesh,
                   compiler_params=_SC_PARAMS, scratch_types=_PASS_SCRATCH)
def _sc_pass_iter(er_hbm, idxn2h, idxc2h, co_hbm, ic_hbm, out_hbm,
                  idxn_v, idxc_v, er0, er1, co0, co1, ic0, ic1, w0, w1,
                  z_v, accum, se0, se1, sg0, sg1, si0, si1, sx0, sx1):
    _sc_pass_body(False, er_hbm, idxn2h, idxc2h, co_hbm, ic_hbm, out_hbm,
                  idxn_v, idxc_v, er0, er1, co0, co1, ic0, ic1, w0, w1,
                  z_v, accum, se0, se1, sg0, sg1, si0, si1, sx0, sx1)


def _full_spec(shape):
    return pl.BlockSpec(shape, lambda i: tuple(0 for _ in shape))


def _mlp(x, w1, b1, w2, b2, wo, bo):
    h = _silu(jnp.dot(x, w1, preferred_element_type=jnp.float32) + b1)
    h = _ln(h)
    h = _silu(jnp.dot(h, w2, preferred_element_type=jnp.float32) + b2)
    h = _ln(h)
    return jnp.dot(h, wo, preferred_element_type=jnp.float32) + bo


def _tc_species_kernel(sp, cb4, cw1, cb1, cw2, cb2, cwo, cbo,
                       nw1, nb1, nw2, nb2, nwo, nbo, ccf_ref, ntab_ref):
    sp_ = sp[...]
    h = _silu(sp_ * cw1[...] + cb1[...])
    h = _ln(h)
    h = _silu(jnp.dot(h, cw2[...], preferred_element_type=jnp.float32) + cb2[...])
    h = _ln(h)
    ccf_ref[...] = jnp.dot(h, cwo[...], preferred_element_type=jnp.float32) + cbo[...]
    g = _silu(sp_ * nw1[...] + nb1[...])
    g = _ln(g)
    g = _silu(jnp.dot(g, nw2[...], preferred_element_type=jnp.float32) + nb2[...])
    g = _ln(g)
    nc = jnp.dot(g, nwo[...], preferred_element_type=jnp.float32) + nbo[...]
    ntab_ref[...] = jnp.concatenate(
        [nc, cb4[...][:, :3], jnp.zeros_like(nc[:, :5])], axis=1)


def _tc_edges_kernel(gtn, gtc, sh, nf, er_ref):
    gt = gtn[...].T
    gc = gtc[...].T
    sht = sh[...].T
    nft = nf[...].T
    dv = gt[24:27, :] - gc[0:3, :] + sht
    x = dv[0:1, :]
    y = dv[1:2, :]
    z = dv[2:3, :]
    r2 = x * x + y * y + z * z
    dist = jnp.sqrt(r2)
    c1 = 0.4886025119029199
    c2 = 1.0925484305920792
    sph = jnp.concatenate([
        jnp.full_like(x, 0.28209479177387814),
        c1 * y, c1 * z, c1 * x,
        c2 * x * y, c2 * y * z,
        0.31539156525252005 * (3.0 * z * z - r2),
        c2 * x * z,
        0.5462742152960396 * (x * x - y * y)], axis=0)
    t = 0.5 * jnp.cos(dist * (np.pi / CUTOFF)) + 0.5
    cut = nft * t * t
    w1 = gt[0:NWAVE, :]
    w2 = gt[NWAVE:2 * NWAVE, :]
    w3 = gt[2 * NWAVE:3 * NWAVE, :]
    radial = jnp.exp(-jnp.square(w2 * (dist - w3)))
    aa = cut * radial * w1
    z7 = jnp.zeros_like(gt[0:7, :])
    er_ref[...] = jnp.concatenate([sph, z7, aa, cut, z7], axis=0).T


def _tc_iter_a0_kernel(p0, p1, w, ccf, co_ref, den_ref):
    co = p0[0] + p1[0]
    co_ref[...] = co
    cf = jnp.dot(co, w[...], preferred_element_type=jnp.float32)
    acc = None
    for k in range(NANG):
        blk = cf[:, k * NCON:(k + 1) * NCON]
        sq = blk * blk
        acc = sq if acc is None else acc + sq
    den_ref[...] = acc * ccf[...]


def _tc_iter_a_kernel(p0, p1, co_prev, w, ccf, den_prev, co_ref, den_ref):
    co = co_prev[...] + p0[0] + p1[0]
    co_ref[...] = co
    cf = jnp.dot(co, w[...], preferred_element_type=jnp.float32)
    acc = None
    for k in range(NANG):
        blk = cf[:, k * NCON:(k + 1) * NCON]
        sq = blk * blk
        acc = sq if acc is None else acc + sq
    den_ref[...] = den_prev[...] + acc * ccf[...]


def _tc_iter_b_kernel(den, w1, b1, w2, b2, wo, bo, it_ref):
    o = _mlp(den[...], w1[...], b1[...], w2[...], b2[...], wo[...], bo[...])
    it_ref[...] = jnp.concatenate([o, jnp.zeros_like(o)], axis=1)


def _tc_final_kernel(den, w1, b1, w2, b2, wo, bo, cf2, out_ref):
    i = pl.program_id(0)
    o = _mlp(den[...], w1[...], b1[...], w2[...], b2[...], wo[...], bo[...])
    part = jnp.sum(o * cf2[...])

    @pl.when(i == 0)
    def _():
        out_ref[...] = jnp.zeros_like(out_ref)

    out_ref[...] += jnp.full((1, 1), 1.0, jnp.float32) * part


BN_N = 2000
BN_E = 2560


def _species_call(sp, cart4, pc, pn):
    specs = [pl.BlockSpec((BN_N, 1), lambda i: (i, 0)),
             pl.BlockSpec((BN_N, 4), lambda i: (i, 0))]
    args = [sp, cart4]
    for p in (pc, pn):
        for nm, bshape in (('W1', None), ('b1', None), ('W2', None),
                           ('b2', None), ('Wo', None), ('bo', None)):
            a = p[nm]
            if a.ndim == 1:
                a = a[None, :]
            specs.append(_full_spec(a.shape))
            args.append(a)
    return pl.pallas_call(
        _tc_species_kernel,
        grid=(N // BN_N,),
        in_specs=specs,
        out_specs=[pl.BlockSpec((BN_N, NCON), lambda i: (i, 0)),
                   pl.BlockSpec((BN_N, ERB), lambda i: (i, 0))],
        out_shape=[jax.ShapeDtypeStruct((N, NCON), jnp.float32),
                   jax.ShapeDtypeStruct((N, ERB), jnp.float32)],
    )(*args)


def _edges_call(gtn, gtc, sh3, nf2):
    return pl.pallas_call(
        _tc_edges_kernel,
        grid=(E // BN_E,),
        in_specs=[pl.BlockSpec((BN_E, ERB), lambda i: (i, 0)),
                  pl.BlockSpec((BN_E, 16), lambda i: (i, 0)),
                  pl.BlockSpec((BN_E, 3), lambda i: (i, 0)),
                  pl.BlockSpec((BN_E, 1), lambda i: (i, 0))],
        out_specs=pl.BlockSpec((BN_E, ERB), lambda i: (i, 0)),
        out_shape=jax.ShapeDtypeStruct((E, ERB), jnp.float32),
    )(gtn, gtc, sh3, nf2)


def _iter_a0_call(p, w, ccf):
    return pl.pallas_call(
        _tc_iter_a0_kernel,
        grid=(N // BN_N,),
        in_specs=[pl.BlockSpec((1, BN_N, ROW), lambda i: (0, i, 0)),
                  pl.BlockSpec((1, BN_N, ROW), lambda i: (1, i, 0)),
                  _full_spec(w.shape),
                  pl.BlockSpec((BN_N, NCON), lambda i: (i, 0))],
        out_specs=[pl.BlockSpec((BN_N, ROW), lambda i: (i, 0)),
                   pl.BlockSpec((BN_N, NCON), lambda i: (i, 0))],
        out_shape=[jax.ShapeDtypeStruct((N, ROW), jnp.float32),
                   jax.ShapeDtypeStruct((N, NCON), jnp.float32)],
    )(p, p, w, ccf)


def _iter_a_call(p, co_prev, w, ccf, den_prev):
    return pl.pallas_call(
        _tc_iter_a_kernel,
        grid=(N // BN_N,),
        in_specs=[pl.BlockSpec((1, BN_N, ROW), lambda i: (0, i, 0)),
                  pl.BlockSpec((1, BN_N, ROW), lambda i: (1, i, 0)),
                  pl.BlockSpec((BN_N, ROW), lambda i: (i, 0)),
                  _full_spec(w.shape),
                  pl.BlockSpec((BN_N, NCON), lambda i: (i, 0)),
                  pl.BlockSpec((BN_N, NCON), lambda i: (i, 0))],
        out_specs=[pl.BlockSpec((BN_N, ROW), lambda i: (i, 0)),
                   pl.BlockSpec((BN_N, NCON), lambda i: (i, 0))],
        out_shape=[jax.ShapeDtypeStruct((N, ROW), jnp.float32),
                   jax.ShapeDtypeStruct((N, NCON), jnp.float32)],
    )(p, p, co_prev, w, ccf, den_prev)


def _iter_b_call(den, pit):
    args = [den]
    specs = [pl.BlockSpec((BN_N, NCON), lambda i: (i, 0))]
    for nm in ('W1', 'b1', 'W2', 'b2', 'Wo', 'bo'):
        a = pit[nm]
        if a.ndim == 1:
            a = a[None, :]
        specs.append(_full_spec(a.shape))
        args.append(a)
    return pl.pallas_call(
        _tc_iter_b_kernel,
        grid=(N // BN_N,),
        in_specs=specs,
        out_specs=pl.BlockSpec((BN_N, 16), lambda i: (i, 0)),
        out_shape=jax.ShapeDtypeStruct((N, 16), jnp.float32),
    )(*args)


def _final_call(den, pout, cf2):
    args = [den]
    specs = [pl.BlockSpec((BN_N, NCON), lambda i: (i, 0))]
    for nm in ('W1', 'b1', 'W2', 'b2', 'Wo', 'bo'):
        a = pout[nm]
        if a.ndim == 1:
            a = a[None, :]
        specs.append(_full_spec(a.shape))
        args.append(a)
    args.append(cf2)
    specs.append(pl.BlockSpec((BN_N, 1), lambda i: (i, 0)))
    return pl.pallas_call(
        _tc_final_kernel,
        grid=(N // BN_N,),
        in_specs=specs,
        out_specs=pl.BlockSpec((1, 1), lambda i: (0, 0)),
        out_shape=jax.ShapeDtypeStruct((1, 1), jnp.float32),
    )(*args)


def kernel(cart, neighlist, shifts, center_factor, neigh_factor, species, params):
    f32 = jnp.float32
    idx_c = neighlist[0].astype(jnp.int32)
    idx_n = neighlist[1].astype(jnp.int32)
    idxn2h = idx_n.reshape(NWK * NBATCH, B)
    idxc2h = idx_c.reshape(NWK * NBATCH, B)
    cart4 = jnp.pad(cart.astype(f32), ((0, 0), (0, 1)))
    cart16 = jnp.pad(cart.astype(f32), ((0, 0), (0, 13)))
    nf2 = neigh_factor.astype(f32)[:, None]
    cf2 = center_factor.astype(f32)[:, None]

    ccf, ntab = _species_call(species.astype(f32), cart4,
                              params['center'], params['neigh'])
    gtn, gtc = _sc_gather(ntab, cart16, idxn2h, idxc2h)
    er = _edges_call(gtn, gtc, shifts.astype(f32), nf2)

    cc_full = params['contracted_coeff'][:, INDEX_L]
    ws = []
    for t in range(ITER_LOOP + 1):
        w = jnp.zeros((ROW, NANG * NCON), f32)
        for k in range(NANG):
            w = w.at[k * NWAVE:(k + 1) * NWAVE, k * NCON:(k + 1) * NCON].set(cc_full[t, k])
        ws.append(w)

    p = _sc_pass0(er, idxc2h)
    co, den = _iter_a0_call(p, ws[0], ccf)
    for t in range(ITER_LOOP):
        it16 = _iter_b_call(den, params['iter'][t])
        p = _sc_pass_iter(er, idxn2h, idxc2h, co, it16)
        co, den = _iter_a_call(p, co, ws[t + 1], ccf, den)
    res = _final_call(den, params['out'], cf2)
    return res[0, 0]


# speedup vs baseline: 72.3478x; 1.1621x over previous
import functools

import jax
import jax.numpy as jnp
import numpy as np
from jax import lax
from jax.experimental import pallas as pl
from jax.experimental.pallas import tpu as pltpu
from jax.experimental.pallas import tpu_sc as plsc

N = 10000
E = 320000
NWAVE = 8
NANG = 9
NCON = 64
CUTOFF = 4.0
ITER_LOOP = 3
INDEX_L = np.array([0, 1, 1, 1, 2, 2, 2, 2, 2])

ROW = 80
ERB = 32
KCH = ROW // 16

NC_ = 2
NS_ = 16
NWK = NC_ * NS_
PER_W = E // NWK
B = 80
NBATCH = PER_W // B
RPT = N // NS_

_mesh = plsc.VectorSubcoreMesh(core_axis_name="c", subcore_axis_name="s")


def _silu(x):
    return x * jax.nn.sigmoid(x)


def _ln(x):
    m = jnp.mean(x, axis=-1, keepdims=True)
    v = jnp.var(x, axis=-1, keepdims=True)
    return (x - m) / jnp.sqrt(v + 1e-5)


_SC_PARAMS = pltpu.CompilerParams(use_tc_tiling_on_sc=False,
                                  needs_layout_passes=False)


@functools.partial(
    pl.kernel,
    compiler_params=_SC_PARAMS,
    out_type=[
        jax.ShapeDtypeStruct((E, ERB), jnp.float32),
        jax.ShapeDtypeStruct((E, ERB), jnp.float32),
    ],
    mesh=_mesh,
    scratch_types=[
        pltpu.VMEM((PER_W,), jnp.int32),
        pltpu.VMEM((PER_W,), jnp.int32),
        pltpu.VMEM((PER_W,), jnp.float32),
        pltpu.VMEM((B, 3), jnp.float32),
        pltpu.VMEM((B, 3), jnp.float32),
        pltpu.VMEM((B, ERB), jnp.float32),
        pltpu.VMEM((B, ERB), jnp.float32),
        pltpu.VMEM((B, ERB), jnp.float32),
        pltpu.VMEM((B, ERB), jnp.float32),
        pltpu.SemaphoreType.DMA,
        pltpu.SemaphoreType.DMA,
        pltpu.SemaphoreType.DMA,
        pltpu.SemaphoreType.DMA,
        pltpu.SemaphoreType.DMA,
        pltpu.SemaphoreType.DMA,
        pltpu.SemaphoreType.DMA,
        pltpu.SemaphoreType.DMA,
    ],
)
def _sc_gather(ntab, cart32, nl, nf1, sh3, gtn_out, gtc_out,
               idxn_v, idxc_v, nf_v, sh0, sh1, gtn0, gtn1, gtc0, gtc1,
               sg0, sg1, sc0, sc1, sh0s, sh1s, sw0, sw1):
    c = lax.axis_index("c")
    s = lax.axis_index("s")
    wid = s * NC_ + c
    pltpu.sync_copy(nl.at[1, pl.ds(wid * PER_W, PER_W)], idxn_v)
    pltpu.sync_copy(nl.at[0, pl.ds(wid * PER_W, PER_W)], idxc_v)
    pltpu.sync_copy(nf1.at[pl.ds(wid * PER_W, PER_W)], nf_v)
    gtn = (gtn0, gtn1)
    gtc = (gtc0, gtc1)
    shv = (sh0, sh1)
    sg = (sg0, sg1)
    sc = (sc0, sc1)
    shs = (sh0s, sh1s)
    sw = (sw0, sw1)
    lane = lax.iota(jnp.int32, 16)
    col3 = jnp.full((16,), 3, jnp.int32)

    def issue(b, k):
        base = wid * PER_W + b * B
        pltpu.async_copy(ntab.at[idxn_v.at[pl.ds(b * B, B)]], gtn[k], sg[k])
        pltpu.async_copy(cart32.at[idxc_v.at[pl.ds(b * B, B)]], gtc[k], sc[k])
        pltpu.async_copy(sh3.at[pl.ds(base, B)], shv[k], shs[k])

    def process(b, k):
        base = wid * PER_W + b * B
        pltpu.make_async_copy(ntab.at[idxn_v.at[pl.ds(b * B, B)]],
                              gtn[k], sg[k]).wait()
        pltpu.make_async_copy(cart32.at[idxc_v.at[pl.ds(b * B, B)]],
                              gtc[k], sc[k]).wait()
        pltpu.make_async_copy(sh3.at[pl.ds(base, B)], shv[k], shs[k]).wait()
        for q in range(B // 16):
            e16 = lane + q * 16
            v = nf_v[pl.ds(b * B + q * 16, 16)]
            plsc.store_scatter(gtc[k], [e16, col3], v)
            for d in range(3):
                sv = plsc.load_gather(
                    shv[k], [e16, jnp.full((16,), d, jnp.int32)])
                plsc.store_scatter(
                    gtc[k], [e16, jnp.full((16,), 4 + d, jnp.int32)], sv)
        pltpu.async_copy(gtn[k], gtn_out.at[pl.ds(base, B)], sw[k])
        pltpu.async_copy(gtc[k], gtc_out.at[pl.ds(base, B)], sw[k])
        pltpu.make_async_copy(gtn[k], gtn_out.at[pl.ds(base, B)],
                              sw[k]).wait()
        pltpu.make_async_copy(gtc[k], gtc_out.at[pl.ds(base, B)],
                              sw[k]).wait()

        @pl.when(b + 2 < NBATCH)
        def _():
            issue(b + 2, k)

    issue(0, 0)
    issue(1, 1)

    def pair(g, carry):
        process(2 * g, 0)
        process(2 * g + 1, 1)
        return carry

    lax.fori_loop(0, NBATCH // 2, pair, 0)
    process(NBATCH - 1, 0)


def _sc_pass_body(first, er_hbm, nl, co_hbm, ic_hbm, out_hbm,
                  idxn_v, idxc1_v, idxc_v, er0, er1, co0, co1, ic0, ic1,
                  w0, w1, z_v, accum, se0, se1, sg0, sg1, si0, si1, sx0, sx1):
    c = lax.axis_index("c")
    s = lax.axis_index("s")
    wid = s * NC_ + c
    lane = lax.iota(jnp.int32, 16)
    j8 = lane & 7
    aa_idx = j8 + 16
    sph_idx = [(ci * 16 + lane) >> 3 for ci in range(KCH)]
    cut_idx = jnp.full((16,), 24, jnp.int32)
    zeros16 = jnp.zeros((16,), jnp.float32)
    er = (er0, er1)
    co = (co0, co1)
    ic = (ic0, ic1)
    w = (w0, w1)
    se = (se0, se1)
    sg = (sg0, sg1)
    si = (si0, si1)
    sx = (sx0, sx1)

    pltpu.sync_copy(nl.at[0, pl.ds(wid * PER_W, PER_W)], idxc1_v)
    if not first:
        pltpu.sync_copy(nl.at[1, pl.ds(wid * PER_W, PER_W)], idxn_v)

    def rpk(i, carry):
        r = i // (B // 16)
        q = i - r * (B // 16)
        idxc_v[r, pl.ds(q * 16, 16)] = idxc1_v[pl.ds(i * 16, 16)]
        return carry

    lax.fori_loop(0, NBATCH * (B // 16), rpk, 0)

    def zrow(i, carry):
        for ci in range(KCH):
            z_v[i, pl.ds(ci * 16, 16)] = zeros16
        return carry

    lax.fori_loop(0, RPT // 5, zrow, 0)
    for k in range(5):
        pltpu.sync_copy(z_v, accum.at[pl.ds(s * RPT + k * (RPT // 5), RPT // 5)])
    plsc.subcore_barrier()

    def issue(b, k):
        base = wid * PER_W + b * B
        pltpu.async_copy(er_hbm.at[pl.ds(base // 4, B // 4)], er[k], se[k])
        if not first:
            pltpu.async_copy(co_hbm.at[idxn_v.at[pl.ds(b * B, B)]], co[k], sg[k])
            pltpu.async_copy(ic_hbm.at[idxn_v.at[pl.ds(b * B, B)]], ic[k], si[k])

    def process(b, k):
        base = wid * PER_W + b * B
        pltpu.make_async_copy(er_hbm.at[pl.ds(base // 4, B // 4)],
                              er[k], se[k]).wait()
        if not first:
            pltpu.make_async_copy(co_hbm.at[idxn_v.at[pl.ds(b * B, B)]],
                                  co[k], sg[k]).wait()
            pltpu.make_async_copy(ic_hbm.at[idxn_v.at[pl.ds(b * B, B)]],
                                  ic[k], si[k]).wait()

        @pl.when(b >= 2)
        def _():
            pltpu.make_async_copy(w[k], accum.at[idxc_v.at[b - 2]], sx[k]).wait()

        er_k, co_k, ic_k, w_k = er[k], co[k], ic[k], w[k]

        def edge(e, carry2):
            efull = jnp.full((16,), e, jnp.int32)
            rfull = jnp.full((16,), e >> 2, jnp.int32)
            cb = (e & 3) * 32
            aab = plsc.load_gather(er_k, [rfull, cb + aa_idx])
            if first:
                for ci in range(KCH):
                    sphb = plsc.load_gather(er_k, [rfull, cb + sph_idx[ci]])
                    w_k[e, pl.ds(ci * 16, 16)] = sphb * aab
            else:
                icb = plsc.load_gather(ic_k, [efull, j8])
                bb = icb * aab
                cut = plsc.load_gather(er_k, [rfull, cb + cut_idx])
                for ci in range(KCH):
                    sphb = plsc.load_gather(er_k, [rfull, cb + sph_idx[ci]])
                    cov = co_k[e, pl.ds(ci * 16, 16)]
                    w_k[e, pl.ds(ci * 16, 16)] = bb * sphb + cut * cov
            return carry2

        lax.fori_loop(0, B, edge, 0)
        pltpu.async_copy(w[k], accum.at[idxc_v.at[b]], sx[k], add=True)

        @pl.when(b + 2 < NBATCH)
        def _():
            issue(b + 2, k)

    issue(0, 0)
    issue(1, 1)

    def pair(g, carry):
        process(2 * g, 0)
        process(2 * g + 1, 1)
        return carry

    lax.fori_loop(0, NBATCH // 2, pair, 0)
    process(NBATCH - 1, 0)
    pltpu.make_async_copy(w[0], accum.at[idxc_v.at[NBATCH - 1]], sx[0]).wait()
    pltpu.make_async_copy(w[1], accum.at[idxc_v.at[NBATCH - 2]], sx[1]).wait()
    plsc.subcore_barrier()
    for k in range(5):
        lo = s * RPT + k * (RPT // 5)
        pltpu.sync_copy(accum.at[pl.ds(lo, RPT // 5)],
                        out_hbm.at[c, pl.ds(lo, RPT // 5)])


_PASS_SCRATCH = [
    pltpu.VMEM((PER_W,), jnp.int32),
    pltpu.VMEM((PER_W,), jnp.int32),
    pltpu.VMEM((NBATCH, B), jnp.int32),
    pltpu.VMEM((B // 4, 128), jnp.float32),
    pltpu.VMEM((B // 4, 128), jnp.float32),
    pltpu.VMEM((B, ROW), jnp.float32),
    pltpu.VMEM((B, ROW), jnp.float32),
    pltpu.VMEM((B, 16), jnp.float32),
    pltpu.VMEM((B, 16), jnp.float32),
    pltpu.VMEM((B, ROW), jnp.float32),
    pltpu.VMEM((B, ROW), jnp.float32),
    pltpu.VMEM((RPT // 5, ROW), jnp.float32),
    pltpu.VMEM_SHARED((N, ROW), jnp.float32),
] + [pltpu.SemaphoreType.DMA] * 8

_OUT_PARTIAL = jax.ShapeDtypeStruct((NC_, N, ROW), jnp.float32)


@functools.partial(pl.kernel, out_type=_OUT_PARTIAL, mesh=_mesh,
                   compiler_params=_SC_PARAMS, scratch_types=_PASS_SCRATCH)
def _sc_pass0(er_hbm, nl, out_hbm,
              idxn_v, idxc1_v, idxc_v, er0, er1, co0, co1, ic0, ic1, w0, w1,
              z_v, accum, se0, se1, sg0, sg1, si0, si1, sx0, sx1):
    _sc_pass_body(True, er_hbm, nl, None, None, out_hbm,
                  idxn_v, idxc1_v, idxc_v, er0, er1, co0, co1, ic0, ic1,
                  w0, w1, z_v, accum, se0, se1, sg0, sg1, si0, si1, sx0, sx1)


@functools.partial(pl.kernel, out_type=_OUT_PARTIAL, mesh=_mesh,
                   compiler_params=_SC_PARAMS, scratch_types=_PASS_SCRATCH)
def _sc_pass_iter(er_hbm, nl, co_hbm, ic_hbm, out_hbm,
                  idxn_v, idxc1_v, idxc_v, er0, er1, co0, co1, ic0, ic1,
                  w0, w1, z_v, accum, se0, se1, sg0, sg1, si0, si1, sx0, sx1):
    _sc_pass_body(False, er_hbm, nl, co_hbm, ic_hbm, out_hbm,
                  idxn_v, idxc1_v, idxc_v, er0, er1, co0, co1, ic0, ic1,
                  w0, w1, z_v, accum, se0, se1, sg0, sg1, si0, si1, sx0, sx1)


def _full_spec(shape):
    return pl.BlockSpec(shape, lambda i: tuple(0 for _ in shape))


def _mlp(x, w1, b1, w2, b2, wo, bo):
    h = _silu(jnp.dot(x, w1, preferred_element_type=jnp.float32) + b1)
    h = _ln(h)
    h = _silu(jnp.dot(h, w2, preferred_element_type=jnp.float32) + b2)
    h = _ln(h)
    return jnp.dot(h, wo, preferred_element_type=jnp.float32) + bo


def _tc_species_kernel(sp, cb3, cw1, cb1, cw2, cb2, cwo, cbo,
                       nw1, nb1, nw2, nb2, nwo, nbo,
                       ccf_ref, ntab_ref, c32_ref):
    sp_ = sp[...]
    h = _silu(sp_ * cw1[...] + cb1[...])
    h = _ln(h)
    h = _silu(jnp.dot(h, cw2[...], preferred_element_type=jnp.float32) + cb2[...])
    h = _ln(h)
    ccf_ref[...] = jnp.dot(h, cwo[...], preferred_element_type=jnp.float32) + cbo[...]
    g = _silu(sp_ * nw1[...] + nb1[...])
    g = _ln(g)
    g = _silu(jnp.dot(g, nw2[...], preferred_element_type=jnp.float32) + nb2[...])
    g = _ln(g)
    nc = jnp.dot(g, nwo[...], preferred_element_type=jnp.float32) + nbo[...]
    cb = cb3[...]
    ntab_ref[...] = jnp.concatenate(
        [nc, cb, jnp.zeros_like(nc[:, :5])], axis=1)
    c32_ref[...] = jnp.concatenate(
        [cb, jnp.zeros_like(nc), jnp.zeros_like(nc[:, :5])], axis=1)


def _tc_edges_kernel(gtn, gtc, er_ref):
    bn4 = BN_E // 4
    g3 = gtn[...].T.reshape(4, ERB, bn4)
    c3 = gtc[...].T.reshape(4, ERB, bn4)
    x = g3[:, 24, :] - c3[:, 0, :] + c3[:, 4, :]
    y = g3[:, 25, :] - c3[:, 1, :] + c3[:, 5, :]
    z = g3[:, 26, :] - c3[:, 2, :] + c3[:, 6, :]
    r2 = x * x + y * y + z * z
    dist = jnp.sqrt(r2)
    c1 = 0.4886025119029199
    c2 = 1.0925484305920792
    sph = jnp.concatenate([
        s[:, None, :] for s in (
            jnp.full_like(x, 0.28209479177387814),
            c1 * y, c1 * z, c1 * x,
            c2 * x * y, c2 * y * z,
            0.31539156525252005 * (3.0 * z * z - r2),
            c2 * x * z,
            0.5462742152960396 * (x * x - y * y))], axis=1)
    t = 0.5 * jnp.cos(dist * (np.pi / CUTOFF)) + 0.5
    cut = c3[:, 3, :] * t * t
    w1 = g3[:, 0:NWAVE, :]
    w2 = g3[:, NWAVE:2 * NWAVE, :]
    w3 = g3[:, 2 * NWAVE:3 * NWAVE, :]
    radial = jnp.exp(-jnp.square(w2 * (dist[:, None, :] - w3)))
    aa = cut[:, None, :] * radial * w1
    z7 = jnp.zeros_like(sph[:, 0:7, :])
    erp = jnp.concatenate([sph, z7, aa, cut[:, None, :], z7], axis=1)
    er_ref[...] = erp.reshape(128, bn4).T


def _tc_iter_a0_kernel(p0, p1, w, ccf, co_ref, den_ref):
    co = p0[0] + p1[0]
    co_ref[...] = co
    cf = jnp.dot(co, w[...], preferred_element_type=jnp.float32)
    acc = None
    for k in range(NANG):
        blk = cf[:, k * NCON:(k + 1) * NCON]
        sq = blk * blk
        acc = sq if acc is None else acc + sq
    den_ref[...] = acc * ccf[...]


def _tc_iter_a_kernel(p0, p1, co_prev, w, ccf, den_prev, co_ref, den_ref):
    co = co_prev[...] + p0[0] + p1[0]
    co_ref[...] = co
    cf = jnp.dot(co, w[...], preferred_element_type=jnp.float32)
    acc = None
    for k in range(NANG):
        blk = cf[:, k * NCON:(k + 1) * NCON]
        sq = blk * blk
        acc = sq if acc is None else acc + sq
    den_ref[...] = den_prev[...] + acc * ccf[...]


def _tc_iter_b_kernel(den, w1, b1, w2, b2, wo, bo, it_ref):
    o = _mlp(den[...], w1[...], b1[...], w2[...], b2[...], wo[...], bo[...])
    it_ref[...] = jnp.concatenate([o, jnp.zeros_like(o)], axis=1)


def _tc_final_kernel(den, w1, b1, w2, b2, wo, bo, cf2, out_ref):
    i = pl.program_id(0)
    o = _mlp(den[...], w1[...], b1[...], w2[...], b2[...], wo[...], bo[...])
    part = jnp.sum(o * cf2[...])

    @pl.when(i == 0)
    def _():
        out_ref[...] = jnp.zeros_like(out_ref)

    out_ref[...] += jnp.full((1, 1), 1.0, jnp.float32) * part


BN_N = 2000
BN_E = 2560


def _species_call(sp, cart, pc, pn):
    specs = [pl.BlockSpec((BN_N, 1), lambda i: (i, 0)),
             pl.BlockSpec((BN_N, 3), lambda i: (i, 0))]
    args = [sp, cart]
    for p in (pc, pn):
        for nm in ('W1', 'b1', 'W2', 'b2', 'Wo', 'bo'):
            a = p[nm]
            if a.ndim == 1:
                a = a[None, :]
            specs.append(_full_spec(a.shape))
            args.append(a)
    return pl.pallas_call(
        _tc_species_kernel,
        grid=(N // BN_N,),
        in_specs=specs,
        out_specs=[pl.BlockSpec((BN_N, NCON), lambda i: (i, 0)),
                   pl.BlockSpec((BN_N, ERB), lambda i: (i, 0)),
                   pl.BlockSpec((BN_N, ERB), lambda i: (i, 0))],
        out_shape=[jax.ShapeDtypeStruct((N, NCON), jnp.float32),
                   jax.ShapeDtypeStruct((N, ERB), jnp.float32),
                   jax.ShapeDtypeStruct((N, ERB), jnp.float32)],
    )(*args)


def _edges_call(gtn, gtc):
    return pl.pallas_call(
        _tc_edges_kernel,
        grid=(E // BN_E,),
        in_specs=[pl.BlockSpec((BN_E // 4, 128), lambda i: (i, 0)),
                  pl.BlockSpec((BN_E // 4, 128), lambda i: (i, 0))],
        out_specs=pl.BlockSpec((BN_E // 4, 128), lambda i: (i, 0)),
        out_shape=jax.ShapeDtypeStruct((E // 4, 128), jnp.float32),
    )(gtn, gtc)


def _iter_a0_call(p, w, ccf):
    return pl.pallas_call(
        _tc_iter_a0_kernel,
        grid=(N // BN_N,),
        in_specs=[pl.BlockSpec((1, BN_N, ROW), lambda i: (0, i, 0)),
                  pl.BlockSpec((1, BN_N, ROW), lambda i: (1, i, 0)),
                  _full_spec(w.shape),
                  pl.BlockSpec((BN_N, NCON), lambda i: (i, 0))],
        out_specs=[pl.BlockSpec((BN_N, ROW), lambda i: (i, 0)),
                   pl.BlockSpec((BN_N, NCON), lambda i: (i, 0))],
        out_shape=[jax.ShapeDtypeStruct((N, ROW), jnp.float32),
                   jax.ShapeDtypeStruct((N, NCON), jnp.float32)],
    )(p, p, w, ccf)


def _iter_a_call(p, co_prev, w, ccf, den_prev):
    return pl.pallas_call(
        _tc_iter_a_kernel,
        grid=(N // BN_N,),
        in_specs=[pl.BlockSpec((1, BN_N, ROW), lambda i: (0, i, 0)),
                  pl.BlockSpec((1, BN_N, ROW), lambda i: (1, i, 0)),
                  pl.BlockSpec((BN_N, ROW), lambda i: (i, 0)),
                  _full_spec(w.shape),
                  pl.BlockSpec((BN_N, NCON), lambda i: (i, 0)),
                  pl.BlockSpec((BN_N, NCON), lambda i: (i, 0))],
        out_specs=[pl.BlockSpec((BN_N, ROW), lambda i: (i, 0)),
                   pl.BlockSpec((BN_N, NCON), lambda i: (i, 0))],
        out_shape=[jax.ShapeDtypeStruct((N, ROW), jnp.float32),
                   jax.ShapeDtypeStruct((N, NCON), jnp.float32)],
    )(p, p, co_prev, w, ccf, den_prev)


def _iter_b_call(den, pit):
    args = [den]
    specs = [pl.BlockSpec((BN_N, NCON), lambda i: (i, 0))]
    for nm in ('W1', 'b1', 'W2', 'b2', 'Wo', 'bo'):
        a = pit[nm]
        if a.ndim == 1:
            a = a[None, :]
        specs.append(_full_spec(a.shape))
        args.append(a)
    return pl.pallas_call(
        _tc_iter_b_kernel,
        grid=(N // BN_N,),
        in_specs=specs,
        out_specs=pl.BlockSpec((BN_N, 16), lambda i: (i, 0)),
        out_shape=jax.ShapeDtypeStruct((N, 16), jnp.float32),
    )(*args)


def _final_call(den, pout, cf2):
    args = [den]
    specs = [pl.BlockSpec((BN_N, NCON), lambda i: (i, 0))]
    for nm in ('W1', 'b1', 'W2', 'b2', 'Wo', 'bo'):
        a = pout[nm]
        if a.ndim == 1:
            a = a[None, :]
        specs.append(_full_spec(a.shape))
        args.append(a)
    args.append(cf2)
    specs.append(pl.BlockSpec((BN_N, 1), lambda i: (i, 0)))
    return pl.pallas_call(
        _tc_final_kernel,
        grid=(N // BN_N,),
        in_specs=specs,
        out_specs=pl.BlockSpec((1, 1), lambda i: (0, 0)),
        out_shape=jax.ShapeDtypeStruct((1, 1), jnp.float32),
    )(*args)


def kernel(cart, neighlist, shifts, center_factor, neigh_factor, species, params):
    f32 = jnp.float32
    nl = neighlist.astype(jnp.int32)
    cf2 = center_factor.astype(f32)[:, None]

    ccf, ntab, cart32 = _species_call(species.astype(f32), cart.astype(f32),
                                      params['center'], params['neigh'])
    gtn, gtc = _sc_gather(ntab, cart32, nl, neigh_factor.astype(f32),
                          shifts.astype(f32))
    er = _edges_call(gtn.reshape(E // 4, 128), gtc.reshape(E // 4, 128))

    cc_full = params['contracted_coeff'][:, INDEX_L]
    ws = []
    for t in range(ITER_LOOP + 1):
        w = jnp.zeros((ROW, NANG * NCON), f32)
        for k in range(NANG):
            w = w.at[k * NWAVE:(k + 1) * NWAVE, k * NCON:(k + 1) * NCON].set(cc_full[t, k])
        ws.append(w)

    p = _sc_pass0(er, nl)
    co, den = _iter_a0_call(p, ws[0], ccf)
    for t in range(ITER_LOOP):
        it16 = _iter_b_call(den, params['iter'][t])
        p = _sc_pass_iter(er, nl, co, it16)
        co, den = _iter_a_call(p, co, ws[t + 1], ccf, den)
    res = _final_call(den, params['out'], cf2)
    return res[0, 0]
